# Initial kernel scaffold; baseline (speedup 1.0000x reference)
import functools

import jax
import jax.numpy as jnp
from jax import lax
from jax.experimental import pallas as pl
from jax.experimental.pallas import tpu as pltpu
from jax.experimental.pallas import tpu_sc as plsc

N = 2048
HID = 128
K = 16
BLK = 256
GRID = N // BLK
LOG2PI = 1.8378770664093453


def _row(d):
    return pl.BlockSpec((BLK, d), lambda i: (i, 0))


def _full(*shape):
    return pl.BlockSpec(shape, lambda i: tuple(0 for _ in shape))


_KMAJ = pl.BlockSpec((K, BLK, HID), lambda i: (0, i, 0))
_F32 = jnp.float32


def _gather_rows(table, idx):
    mesh = plsc.VectorSubcoreMesh(core_axis_name="c", subcore_axis_name="s")
    n_out = idx.shape[0]
    per_w = n_out // 32
    n_chunk = per_w // 128

    @functools.partial(
        pl.kernel, mesh=mesh,
        out_type=jax.ShapeDtypeStruct((n_out, HID), _F32),
        scratch_types=[
            pltpu.VMEM((128,), jnp.int32),
            pltpu.VMEM((128, HID), _F32),
            pltpu.SemaphoreType.DMA,
        ],
    )
    def k(table_hbm, idx_hbm, out_hbm, idx_v, rows_v, sem):
        wid = lax.axis_index("s") * 2 + lax.axis_index("c")
        base = wid * per_w
        for c in range(n_chunk):
            off = base + c * 128
            pltpu.sync_copy(idx_hbm.at[pl.ds(off, 128)], idx_v)
            pltpu.async_copy(table_hbm.at[idx_v], rows_v, sem).wait()
            pltpu.sync_copy(rows_v, out_hbm.at[pl.ds(off, 128)])

    return k(table, idx)


def _make_knn(ne):
    def body(*refs):
        pos_ref, posT_ref, rot_ref = refs[0], refs[1], refs[2]
        we_refs = refs[3:3 + 2 * ne]
        nbr_ref = refs[3 + 2 * ne]
        e_refs = refs[4 + 2 * ne:]
        px = pos_ref[:, 0:1]
        py = pos_ref[:, 1:2]
        qx = posT_ref[0:1, :]
        qy = posT_ref[1:2, :]
        dx = qx - px
        dy = qy - py
        dist = jnp.sqrt(dx * dx + dy * dy + 1e-9)
        jj = lax.broadcasted_iota(jnp.int32, dist.shape, 1)
        r00 = rot_ref[:, 0:1]
        r01 = rot_ref[:, 1:2]
        r10 = rot_ref[:, 2:3]
        r11 = rot_ref[:, 3:4]
        idx_cols = []
        for k in range(K):
            m = jnp.min(dist, axis=1, keepdims=True)
            idx = jnp.min(jnp.where(dist == m, jj, N), axis=1, keepdims=True)
            sel = jj == idx
            dxk = jnp.sum(jnp.where(sel, dx, 0.0), axis=1, keepdims=True)
            dyk = jnp.sum(jnp.where(sel, dy, 0.0), axis=1, keepdims=True)
            dist = jnp.where(sel, jnp.inf, dist)
            idx_cols.append(idx)
            rx = dxk * r00 + dyk * r10
            ry = dxk * r01 + dyk * r11
            for t in range(ne):
                we = we_refs[2 * t][...]
                be = we_refs[2 * t + 1][...]
                e_refs[t][k] = jnp.maximum(
                    rx * we[0:1, :] + ry * we[1:2, :] + be, 0.0)
        nbr_ref[...] = jnp.concatenate(idx_cols, axis=1)

    in_specs = [_row(2), _full(2, N), _row(4)]
    for _ in range(ne):
        in_specs += [_full(2, HID), _full(1, HID)]
    out_specs = [pl.BlockSpec((BLK, K), lambda i: (i, 0))] + [_KMAJ] * ne
    out_shape = [jax.ShapeDtypeStruct((N, K), jnp.int32)] + [
        jax.ShapeDtypeStruct((K, N, HID), _F32)] * ne
    return pl.pallas_call(
        body, grid=(GRID,), in_specs=in_specs, out_specs=out_specs,
        out_shape=out_shape)


def _encin_body(x_ref, wi_ref, bi_ref, wm_ref, h_ref, y_ref):
    h = jnp.maximum(
        jnp.dot(x_ref[...], wi_ref[...], preferred_element_type=_F32)
        + bi_ref[...], 0.0)
    h_ref[...] = h
    y_ref[...] = jnp.dot(h, wm_ref[...], preferred_element_type=_F32)


def _encin(x, wi, bi, wm):
    din = x.shape[1]
    return pl.pallas_call(
        _encin_body, grid=(GRID,),
        in_specs=[_row(din), _full(din, HID), _full(1, HID),
                  _full(HID, HID)],
        out_specs=[_row(HID), _row(HID)],
        out_shape=[jax.ShapeDtypeStruct((N, HID), _F32),
                   jax.ShapeDtypeStruct((N, HID), _F32)],
    )(x, wi, bi.reshape(1, HID), wm)


def _layer_body(h_ref, g3_ref, e3_ref, wb_ref, bm_ref, wu_ref, bu_ref,
                wn_ref, bn_ref, h_out_ref, y_ref):
    e = e3_ref[...].reshape(K * BLK, HID)
    g = g3_ref[...].reshape(K * BLK, HID)
    eb = jnp.dot(e, wb_ref[...], preferred_element_type=_F32)
    msg = jnp.maximum(g + eb + bm_ref[...], 0.0)
    m3 = msg.reshape(K, BLK, HID)
    acc = m3[0]
    for k in range(1, K):
        acc = acc + m3[k]
    agg = acc * (1.0 / K)
    h = h_ref[...]
    u = jnp.maximum(
        jnp.dot(h, wu_ref[:HID, :], preferred_element_type=_F32)
        + jnp.dot(agg, wu_ref[HID:, :], preferred_element_type=_F32)
        + bu_ref[...], 0.0)
    hn = h + u
    h_out_ref[...] = hn
    y_ref[...] = jnp.dot(hn, wn_ref[...], preferred_element_type=_F32) \
        + bn_ref[...]


def _layer(h, g3, e3, w_msg_bot, b_msg, w_upd, b_upd, w_next, b_next):
    dout = w_next.shape[1]
    return pl.pallas_call(
        _layer_body, grid=(GRID,),
        in_specs=[_row(HID), _KMAJ, _KMAJ, _full(HID, HID), _full(1, HID),
                  _full(2 * HID, HID), _full(1, HID), _full(HID, dout),
                  _full(1, dout)],
        out_specs=[_row(HID), _row(dout)],
        out_shape=[jax.ShapeDtypeStruct((N, HID), _F32),
                   jax.ShapeDtypeStruct((N, dout), _F32)],
    )(h, g3, e3, w_msg_bot, b_msg.reshape(1, HID), w_upd,
      b_upd.reshape(1, HID), w_next, b_next.reshape(1, dout))


def _decin_body(he_ref, eps_ref, ox_ref, oy_ref, at_ref, wz_ref, wx_ref,
                wy_ref, wa_ref, b_ref, wm_ref, h_ref, y_ref, klp_ref):
    he = he_ref[...]
    mu = he[:, :64]
    lv = he[:, 64:]
    elv = jnp.exp(lv)
    z = mu + eps_ref[...] * jnp.exp(lv * 0.5)
    klv = jnp.sum(elv + mu * mu - lv - 1.0)
    klp_ref[...] = jnp.broadcast_to(klv.reshape(1, 1, 1), (1, 1, HID))
    h0 = jnp.maximum(
        jnp.dot(z, wz_ref[...], preferred_element_type=_F32)
        + jnp.dot(ox_ref[...], wx_ref[...], preferred_element_type=_F32)
        + jnp.dot(oy_ref[...], wy_ref[...], preferred_element_type=_F32)
        + jnp.dot(at_ref[...], wa_ref[...], preferred_element_type=_F32)
        + b_ref[...], 0.0)
    h_ref[...] = h0
    y_ref[...] = jnp.dot(h0, wm_ref[...], preferred_element_type=_F32)


def _decin(he, eps, ox, oy, at, wz, wx, wy, wa, b, wm):
    return pl.pallas_call(
        _decin_body, grid=(GRID,),
        in_specs=[_row(HID), _row(64), _row(64), _row(64), _row(64),
                  _full(64, HID), _full(64, HID), _full(64, HID),
                  _full(64, HID), _full(1, HID), _full(HID, HID)],
        out_specs=[_row(HID), _row(HID),
                   pl.BlockSpec((1, 1, HID), lambda i: (i, 0, 0))],
        out_shape=[jax.ShapeDtypeStruct((N, HID), _F32),
                   jax.ShapeDtypeStruct((N, HID), _F32),
                   jax.ShapeDtypeStruct((GRID, 1, HID), _F32)],
    )(he, eps, ox, oy, at, wz, wx, wy, wa, b.reshape(1, HID), wm)


def _polin_body(rx_ref, ry_ref, at_ref, wx_ref, wy_ref, wa_ref, b_ref,
                wm_ref, h_ref, y_ref):
    h0 = jnp.maximum(
        jnp.dot(rx_ref[...], wx_ref[...], preferred_element_type=_F32)
        + jnp.dot(ry_ref[...], wy_ref[...], preferred_element_type=_F32)
        + jnp.dot(at_ref[...], wa_ref[...], preferred_element_type=_F32)
        + b_ref[...], 0.0)
    h_ref[...] = h0
    y_ref[...] = jnp.dot(h0, wm_ref[...], preferred_element_type=_F32)


def _polin(rx, ry, at, wx, wy, wa, b, wm):
    return pl.pallas_call(
        _polin_body, grid=(GRID,),
        in_specs=[_row(64), _row(64), _row(64), _full(64, HID),
                  _full(64, HID), _full(64, HID), _full(1, HID),
                  _full(HID, HID)],
        out_specs=[_row(HID), _row(HID)],
        out_shape=[jax.ShapeDtypeStruct((N, HID), _F32),
                   jax.ShapeDtypeStruct((N, HID), _F32)],
    )(rx, ry, at, wx, wy, wa, b.reshape(1, HID), wm)


def _mid_body(rp0_ref, rp1_ref, rp2_ref, rp3_ref, rp4_ref, ox_ref, oy_ref,
              at_ref, e0_ref, e1_ref, nz_ref, rot_ref, op_ref, np_ref,
              rrx_ref, rry_ref, ro_ref, rrot_ref, rnp_ref, lp_ref):
    ox = ox_ref[...]
    oy = oy_ref[...]
    at = at_ref[...]
    mx = rp0_ref[...]
    my = rp1_ref[...]
    off = rp2_ref[...]
    lp3 = rp3_ref[...]
    lp4 = rp4_ref[...]
    d0 = jnp.exp(lp3)
    d1 = jnp.exp(lp4)
    y0 = (ox[:, :10] - mx) / d0
    y1 = (oy[:, :10] - my - off * y0) / d1
    neg_logp = 0.5 * (y0 * y0 + y1 * y1) + lp3 + lp4 + LOG2PI
    mask10 = (at[:, :10] != 0.0).astype(_F32)
    nl_sum = jnp.sum(neg_logp * mask10).reshape(1, 1)
    m_sum = jnp.sum(mask10).reshape(1, 1)
    lp_ref[...] = jnp.concatenate(
        [nl_sum, m_sum, jnp.zeros((1, HID - 2), _F32)], axis=1
    ).reshape(1, 1, HID)
    e0 = e0_ref[...]
    e1 = e1_ref[...]
    s0 = (mx + d0 * e0) * mask10
    s1 = (my + off * e0 + d1 * e1) * mask10
    rsx = jnp.concatenate([s0, ox[:, 10:]], axis=1)
    rsy = jnp.concatenate([s1, oy[:, 10:]], axis=1)
    nx = nz_ref[:, 0:1]
    ny = nz_ref[:, 1:2]
    curx = rsx[:, 0:1] + nx
    cury = rsy[:, 0:1] + ny
    rpx = ox[:, 10:60]
    rpy = oy[:, 10:60]
    wdt = at[:, 10:60]
    ddx = rpx - curx
    ddy = rpy - cury
    avail = (wdt != 0.0).astype(_F32)
    dd = jnp.sqrt(ddx * ddx + ddy * ddy) - wdt - avail * 1000.0
    jj = lax.broadcasted_iota(jnp.int32, dd.shape, 1)
    sx = jnp.zeros(dd.shape, _F32)
    sy = jnp.zeros(dd.shape, _F32)
    sw = jnp.zeros(dd.shape, _F32)
    for i in range(50):
        di = dd[:, i:i + 1]
        less = (dd < di).astype(jnp.int32)
        eq = ((dd == di) & (jj < i)).astype(jnp.int32)
        rank = jnp.sum(less + eq, axis=1, keepdims=True)
        oh = (jj == rank).astype(_F32)
        sx = sx + oh * rpx[:, i:i + 1]
        sy = sy + oh * rpy[:, i:i + 1]
        sw = sw + oh * wdt[:, i:i + 1]
    rsx = jnp.concatenate([rsx[:, :10], sx, rsx[:, 60:]], axis=1)
    rsy = jnp.concatenate([rsy[:, :10], sy, rsy[:, 60:]], axis=1)
    c0 = rot_ref[:, 0:1]
    c1 = rot_ref[:, 1:2]
    c2 = rot_ref[:, 2:3]
    c3 = rot_ref[:, 3:4]
    opx = op_ref[:, 0:1]
    opy = op_ref[:, 1:2]
    absx = rsx * c0 + rsy * c1 + opx
    absy = rsx * c2 + rsy * c3 + opy
    nrx = nx * c0 + ny * c1
    nry = nx * c2 + ny * c3
    rox = absx[:, 0:1] + nrx
    roy = absy[:, 0:1] + nry
    gx = absx[:, 63:64] - rox
    gy = absy[:, 63:64] - roy
    r2 = gx * gx + gy * gy
    inv = lax.rsqrt(r2)
    cc = jnp.where(r2 > 0.0, gx * inv, 1.0)
    ss = jnp.where(r2 > 0.0, gy * inv, 0.0)
    ax = absx - rox
    ay = absy - roy
    rrx = ax * cc + ay * ss
    rry = ay * cc - ax * ss
    recattr = jnp.concatenate([at[:, :10], sw], axis=1)
    mask60 = (recattr != 0.0).astype(_F32)
    rrx = jnp.concatenate([rrx[:, :60] * mask60, rrx[:, 60:]], axis=1)
    rry = jnp.concatenate([rry[:, :60] * mask60, rry[:, 60:]], axis=1)
    rrx_ref[...] = rrx
    rry_ref[...] = rry
    ro_ref[...] = jnp.concatenate([rox, roy], axis=1)
    rrot_ref[...] = jnp.concatenate([cc, -ss, ss, cc], axis=1)
    npx = np_ref[:, 0:1]
    npy = np_ref[:, 1:2]
    rnp_ref[...] = jnp.concatenate(
        [npx + rox - opx, npy + roy - opy], axis=1)


def _mid(rp, ox, oy, at, eps_s, noise, rotf, ori_pos, node_pos):
    return pl.pallas_call(
        _mid_body, grid=(GRID,),
        in_specs=[_row(10)] * 5 + [_row(64)] * 3 + [_row(10)] * 2
        + [_row(2), _row(4), _row(2), _row(2)],
        out_specs=[_row(64), _row(64), _row(2), _row(4), _row(2),
                   pl.BlockSpec((1, 1, HID), lambda i: (i, 0, 0))],
        out_shape=[jax.ShapeDtypeStruct((N, 64), _F32),
                   jax.ShapeDtypeStruct((N, 64), _F32),
                   jax.ShapeDtypeStruct((N, 2), _F32),
                   jax.ShapeDtypeStruct((N, 4), _F32),
                   jax.ShapeDtypeStruct((N, 2), _F32),
                   jax.ShapeDtypeStruct((GRID, 1, HID), _F32)],
    )(rp[..., 0], rp[..., 1], rp[..., 2], rp[..., 3], rp[..., 4],
      ox, oy, at, eps_s[..., 0], eps_s[..., 1], noise, rotf, ori_pos,
      node_pos)


def _graphnet_tail(h, y, e3, idx, p, n_layers, w_out_p, b_out_p):
    for l in range(n_layers):
        g3 = _gather_rows(y, idx).reshape(K, N, HID)
        if l == n_layers - 1:
            w_next, b_next = w_out_p, b_out_p
        else:
            w_next = p['W_msg%d' % (l + 1)][:HID, :]
            b_next = jnp.zeros((w_next.shape[1],), _F32)
        h, y = _layer(h, g3, e3, p['W_msg%d' % l][HID:, :], p['b_msg%d' % l],
                      p['W_upd%d' % l], p['b_upd%d' % l], w_next, b_next)
    return h, y


def _pad_out(w, b, dout):
    dpad = dout if dout % HID == 0 else (dout // HID + 1) * HID
    wp = jnp.zeros((HID, dpad), _F32).at[:, :dout].set(w)
    bp = jnp.zeros((dpad,), _F32).at[:dout].set(b)
    return wp, bp


def kernel(graph_state, node_pos, ori_pos, rotate, enc, dec, pol):
    gs3 = graph_state.reshape(N, 64, 3)
    ox = gs3[:, :, 0]
    oy = gs3[:, :, 1]
    at = gs3[:, :, 2]
    rotf = rotate.reshape(N, 4)
    eps = jax.random.normal(jax.random.key(11), (N, 64), _F32)
    eps_s = jax.random.normal(jax.random.key(12), (N, 10, 2), _F32)
    noise = 0.1 * jax.random.normal(jax.random.key(13), (N, 2), _F32)

    nbr, e3_enc, e3_dec = _make_knn(2)(
        node_pos, node_pos.T, rotf, enc['W_e'], enc['b_e'].reshape(1, HID),
        dec['W_e'], dec['b_e'].reshape(1, HID))
    idx = nbr.T.reshape(-1)

    h, y = _encin(graph_state, enc['W_in'], enc['b_in'],
                  enc['W_msg0'][:HID, :])
    _, y_enc = _graphnet_tail(h, y, e3_enc, idx, enc, 2,
                              enc['W_out'], enc['b_out'])

    wd = dec['W_in']
    wctx = wd[64:172].reshape(54, 2, HID)
    zpad = jnp.zeros((10, HID), _F32)
    wxd = jnp.concatenate([zpad, wctx[:, 0, :]], axis=0)
    wyd = jnp.concatenate([zpad, wctx[:, 1, :]], axis=0)
    h, y, klp = _decin(y_enc, eps, ox, oy, at, wd[:64], wxd, wyd,
                       wd[172:], dec['b_in'], dec['W_msg0'][:HID, :])
    wop, bop = _pad_out(dec['W_out'], dec['b_out'], 50)
    _, y_dec = _graphnet_tail(h, y, e3_dec, idx, dec, 2, wop, bop)
    kl = 0.5 * jnp.sum(klp[:, 0, 0]) / N

    rp = y_dec[:, :50].reshape(N, 10, 5)
    rrx, rry, ro, rrotf, rnp, lossp = _mid(rp, ox, oy, at, eps_s, noise,
                                           rotf, ori_pos, node_pos)
    rec_loss = jnp.sum(lossp[:, 0, 0]) / jnp.maximum(
        jnp.sum(lossp[:, 0, 1]), 1.0)

    nbr_p, e3_pol = _make_knn(1)(
        rnp, rnp.T, rrotf, pol['W_e'], pol['b_e'].reshape(1, HID))
    idx_p = nbr_p.T.reshape(-1)
    wp3 = pol['W_in'].reshape(64, 3, HID)
    h, y = _polin(rrx, rry, at, wp3[:, 0, :], wp3[:, 1, :], wp3[:, 2, :],
                  pol['b_in'], pol['W_msg0'][:HID, :])
    wopp, bopp = _pad_out(pol['W_out'], pol['b_out'], 250)
    _, y_pol = _graphnet_tail(h, y, e3_pol, idx_p, pol, 3, wopp, bopp)
    action_preds = y_pol[:, :250]

    rec_rot = rrotf.reshape(N, 2, 2)
    return action_preds, rec_loss, kl, ro, rec_rot


# speedup vs baseline: 9.0023x; 9.0023x over previous
import functools

import jax
import jax.numpy as jnp
from jax import lax
from jax.experimental import pallas as pl
from jax.experimental.pallas import tpu as pltpu
from jax.experimental.pallas import tpu_sc as plsc

N = 2048
HID = 128
K = 16
BLK = 256
GRID = N // BLK
LOG2PI = 1.8378770664093453


def _row(d):
    return pl.BlockSpec((BLK, d), lambda i: (i, 0))


def _full(*shape):
    return pl.BlockSpec(shape, lambda i: tuple(0 for _ in shape))


_KMAJ = pl.BlockSpec((K, BLK, HID), lambda i: (0, i, 0))
_F32 = jnp.float32


def _bdot(a, b):
    return jnp.dot(a.astype(jnp.bfloat16), b.astype(jnp.bfloat16),
                   preferred_element_type=_F32)


def _bfr(x):
    return x.astype(jnp.bfloat16).astype(_F32)


def _gather_rows(table, idx):
    mesh = plsc.VectorSubcoreMesh(core_axis_name="c", subcore_axis_name="s")
    n_out = idx.shape[0]
    per_w = n_out // 32
    n_chunk = per_w // 128

    @functools.partial(
        pl.kernel, mesh=mesh,
        out_type=jax.ShapeDtypeStruct((n_out, HID), _F32),
        scratch_types=[
            pltpu.VMEM((128,), jnp.int32),
            pltpu.VMEM((128, HID), _F32),
            pltpu.SemaphoreType.DMA,
        ],
    )
    def k(table_hbm, idx_hbm, out_hbm, idx_v, rows_v, sem):
        wid = lax.axis_index("s") * 2 + lax.axis_index("c")
        base = wid * per_w
        for c in range(n_chunk):
            off = base + c * 128
            pltpu.sync_copy(idx_hbm.at[pl.ds(off, 128)], idx_v)
            pltpu.async_copy(table_hbm.at[idx_v], rows_v, sem).wait()
            pltpu.sync_copy(rows_v, out_hbm.at[pl.ds(off, 128)])

    return k(table, idx)


def _make_knn(ne):
    def body(*refs):
        pos_ref, posT_ref, rot_ref = refs[0], refs[1], refs[2]
        we_refs = refs[3:3 + 2 * ne]
        nbr_ref = refs[3 + 2 * ne]
        e_refs = refs[4 + 2 * ne:]
        px = pos_ref[:, 0:1]
        py = pos_ref[:, 1:2]
        qx = posT_ref[0:1, :]
        qy = posT_ref[1:2, :]
        dx = qx - px
        dy = qy - py
        dist = jnp.sqrt(dx * dx + dy * dy + 1e-9)
        jj = lax.broadcasted_iota(jnp.int32, dist.shape, 1)
        r00 = rot_ref[:, 0:1]
        r01 = rot_ref[:, 1:2]
        r10 = rot_ref[:, 2:3]
        r11 = rot_ref[:, 3:4]
        idx_cols = []
        for k in range(K):
            m = jnp.min(dist, axis=1, keepdims=True)
            idx = jnp.min(jnp.where(dist == m, jj, N), axis=1, keepdims=True)
            sel = jj == idx
            dxk = jnp.sum(jnp.where(sel, dx, 0.0), axis=1, keepdims=True)
            dyk = jnp.sum(jnp.where(sel, dy, 0.0), axis=1, keepdims=True)
            dist = jnp.where(sel, jnp.inf, dist)
            idx_cols.append(idx)
            dxb = _bfr(dxk)
            dyb = _bfr(dyk)
            rx = _bfr(dxb * _bfr(r00) + dyb * _bfr(r10))
            ry = _bfr(dxb * _bfr(r01) + dyb * _bfr(r11))
            for t in range(ne):
                we = we_refs[2 * t][...]
                be = we_refs[2 * t + 1][...]
                e_refs[t][k] = jnp.maximum(
                    rx * _bfr(we[0:1, :]) + ry * _bfr(we[1:2, :]) + be, 0.0)
        nbr_ref[...] = jnp.concatenate(idx_cols, axis=1)

    in_specs = [_row(2), _full(2, N), _row(4)]
    for _ in range(ne):
        in_specs += [_full(2, HID), _full(1, HID)]
    out_specs = [pl.BlockSpec((BLK, K), lambda i: (i, 0))] + [_KMAJ] * ne
    out_shape = [jax.ShapeDtypeStruct((N, K), jnp.int32)] + [
        jax.ShapeDtypeStruct((K, N, HID), _F32)] * ne
    return pl.pallas_call(
        body, grid=(GRID,), in_specs=in_specs, out_specs=out_specs,
        out_shape=out_shape)


def _encin_body(x_ref, wi_ref, bi_ref, wm_ref, h_ref, y_ref):
    h = jnp.maximum(
        _bdot(x_ref[...], wi_ref[...])
        + bi_ref[...], 0.0)
    h_ref[...] = h
    y_ref[...] = _bdot(h, wm_ref[...])


def _encin(x, wi, bi, wm):
    din = x.shape[1]
    return pl.pallas_call(
        _encin_body, grid=(GRID,),
        in_specs=[_row(din), _full(din, HID), _full(1, HID),
                  _full(HID, HID)],
        out_specs=[_row(HID), _row(HID)],
        out_shape=[jax.ShapeDtypeStruct((N, HID), _F32),
                   jax.ShapeDtypeStruct((N, HID), _F32)],
    )(x, wi, bi.reshape(1, HID), wm)


def _layer_body(h_ref, g3_ref, e3_ref, wb_ref, bm_ref, wu_ref, bu_ref,
                wn_ref, bn_ref, h_out_ref, y_ref):
    e = e3_ref[...].reshape(K * BLK, HID)
    g = g3_ref[...].reshape(K * BLK, HID)
    eb = _bdot(e, wb_ref[...])
    msg = jnp.maximum(g + eb + bm_ref[...], 0.0)
    m3 = msg.reshape(K, BLK, HID)
    acc = m3[0]
    for k in range(1, K):
        acc = acc + m3[k]
    agg = acc * (1.0 / K)
    h = h_ref[...]
    u = jnp.maximum(
        _bdot(h, wu_ref[:HID, :])
        + _bdot(agg, wu_ref[HID:, :])
        + bu_ref[...], 0.0)
    hn = h + u
    h_out_ref[...] = hn
    y_ref[...] = _bdot(hn, wn_ref[...]) \
        + bn_ref[...]


def _layer(h, g3, e3, w_msg_bot, b_msg, w_upd, b_upd, w_next, b_next):
    dout = w_next.shape[1]
    return pl.pallas_call(
        _layer_body, grid=(GRID,),
        in_specs=[_row(HID), _KMAJ, _KMAJ, _full(HID, HID), _full(1, HID),
                  _full(2 * HID, HID), _full(1, HID), _full(HID, dout),
                  _full(1, dout)],
        out_specs=[_row(HID), _row(dout)],
        out_shape=[jax.ShapeDtypeStruct((N, HID), _F32),
                   jax.ShapeDtypeStruct((N, dout), _F32)],
    )(h, g3, e3, w_msg_bot, b_msg.reshape(1, HID), w_upd,
      b_upd.reshape(1, HID), w_next, b_next.reshape(1, dout))


def _decin_body(he_ref, eps_ref, ox_ref, oy_ref, at_ref, wz_ref, wx_ref,
                wy_ref, wa_ref, b_ref, wm_ref, h_ref, y_ref, klp_ref):
    he = he_ref[...]
    mu = he[:, :64]
    lv = he[:, 64:]
    elv = jnp.exp(lv)
    z = mu + eps_ref[...] * jnp.exp(lv * 0.5)
    klv = jnp.sum(elv + mu * mu - lv - 1.0)
    klp_ref[...] = jnp.broadcast_to(klv.reshape(1, 1, 1), (1, 1, HID))
    h0 = jnp.maximum(
        _bdot(z, wz_ref[...])
        + _bdot(ox_ref[...], wx_ref[...])
        + _bdot(oy_ref[...], wy_ref[...])
        + _bdot(at_ref[...], wa_ref[...])
        + b_ref[...], 0.0)
    h_ref[...] = h0
    y_ref[...] = _bdot(h0, wm_ref[...])


def _decin(he, eps, ox, oy, at, wz, wx, wy, wa, b, wm):
    return pl.pallas_call(
        _decin_body, grid=(GRID,),
        in_specs=[_row(HID), _row(64), _row(64), _row(64), _row(64),
                  _full(64, HID), _full(64, HID), _full(64, HID),
                  _full(64, HID), _full(1, HID), _full(HID, HID)],
        out_specs=[_row(HID), _row(HID),
                   pl.BlockSpec((1, 1, HID), lambda i: (i, 0, 0))],
        out_shape=[jax.ShapeDtypeStruct((N, HID), _F32),
                   jax.ShapeDtypeStruct((N, HID), _F32),
                   jax.ShapeDtypeStruct((GRID, 1, HID), _F32)],
    )(he, eps, ox, oy, at, wz, wx, wy, wa, b.reshape(1, HID), wm)


def _polin_body(rx_ref, ry_ref, at_ref, wx_ref, wy_ref, wa_ref, b_ref,
                wm_ref, h_ref, y_ref):
    h0 = jnp.maximum(
        _bdot(rx_ref[...], wx_ref[...])
        + _bdot(ry_ref[...], wy_ref[...])
        + _bdot(at_ref[...], wa_ref[...])
        + b_ref[...], 0.0)
    h_ref[...] = h0
    y_ref[...] = _bdot(h0, wm_ref[...])


def _polin(rx, ry, at, wx, wy, wa, b, wm):
    return pl.pallas_call(
        _polin_body, grid=(GRID,),
        in_specs=[_row(64), _row(64), _row(64), _full(64, HID),
                  _full(64, HID), _full(64, HID), _full(1, HID),
                  _full(HID, HID)],
        out_specs=[_row(HID), _row(HID)],
        out_shape=[jax.ShapeDtypeStruct((N, HID), _F32),
                   jax.ShapeDtypeStruct((N, HID), _F32)],
    )(rx, ry, at, wx, wy, wa, b.reshape(1, HID), wm)


def _mid_body(rp0_ref, rp1_ref, rp2_ref, rp3_ref, rp4_ref, ox_ref, oy_ref,
              at_ref, e0_ref, e1_ref, nz_ref, rot_ref, op_ref, np_ref,
              rrx_ref, rry_ref, ro_ref, rrot_ref, rnp_ref, lp_ref):
    ox = ox_ref[...]
    oy = oy_ref[...]
    at = at_ref[...]
    mx = rp0_ref[...]
    my = rp1_ref[...]
    off = rp2_ref[...]
    lp3 = rp3_ref[...]
    lp4 = rp4_ref[...]
    d0 = jnp.exp(lp3)
    d1 = jnp.exp(lp4)
    y0 = (ox[:, :10] - mx) / d0
    y1 = (oy[:, :10] - my - off * y0) / d1
    neg_logp = 0.5 * (y0 * y0 + y1 * y1) + lp3 + lp4 + LOG2PI
    mask10 = (at[:, :10] != 0.0).astype(_F32)
    nl_sum = jnp.sum(neg_logp * mask10).reshape(1, 1)
    m_sum = jnp.sum(mask10).reshape(1, 1)
    lp_ref[...] = jnp.concatenate(
        [nl_sum, m_sum, jnp.zeros((1, HID - 2), _F32)], axis=1
    ).reshape(1, 1, HID)
    e0 = e0_ref[...]
    e1 = e1_ref[...]
    s0 = (mx + d0 * e0) * mask10
    s1 = (my + off * e0 + d1 * e1) * mask10
    rsx = jnp.concatenate([s0, ox[:, 10:]], axis=1)
    rsy = jnp.concatenate([s1, oy[:, 10:]], axis=1)
    nx = nz_ref[:, 0:1]
    ny = nz_ref[:, 1:2]
    curx = rsx[:, 0:1] + nx
    cury = rsy[:, 0:1] + ny
    rpx = ox[:, 10:60]
    rpy = oy[:, 10:60]
    wdt = at[:, 10:60]
    ddx = rpx - curx
    ddy = rpy - cury
    avail = (wdt != 0.0).astype(_F32)
    dd = jnp.sqrt(ddx * ddx + ddy * ddy) - wdt - avail * 1000.0
    jj = lax.broadcasted_iota(jnp.int32, dd.shape, 1)
    sx = jnp.zeros(dd.shape, _F32)
    sy = jnp.zeros(dd.shape, _F32)
    sw = jnp.zeros(dd.shape, _F32)
    for i in range(50):
        di = dd[:, i:i + 1]
        less = (dd < di).astype(jnp.int32)
        eq = ((dd == di) & (jj < i)).astype(jnp.int32)
        rank = jnp.sum(less + eq, axis=1, keepdims=True)
        oh = (jj == rank).astype(_F32)
        sx = sx + oh * rpx[:, i:i + 1]
        sy = sy + oh * rpy[:, i:i + 1]
        sw = sw + oh * wdt[:, i:i + 1]
    rsx = jnp.concatenate([rsx[:, :10], sx, rsx[:, 60:]], axis=1)
    rsy = jnp.concatenate([rsy[:, :10], sy, rsy[:, 60:]], axis=1)
    c0 = rot_ref[:, 0:1]
    c1 = rot_ref[:, 1:2]
    c2 = rot_ref[:, 2:3]
    c3 = rot_ref[:, 3:4]
    opx = op_ref[:, 0:1]
    opy = op_ref[:, 1:2]
    rsxb = _bfr(rsx)
    rsyb = _bfr(rsy)
    c0b = _bfr(c0)
    c1b = _bfr(c1)
    c2b = _bfr(c2)
    c3b = _bfr(c3)
    absx = rsxb * c0b + rsyb * c1b + opx
    absy = rsxb * c2b + rsyb * c3b + opy
    nrx = _bfr(nx) * c0b + _bfr(ny) * c1b
    nry = _bfr(nx) * c2b + _bfr(ny) * c3b
    rox = absx[:, 0:1] + nrx
    roy = absy[:, 0:1] + nry
    gx = absx[:, 63:64] - rox
    gy = absy[:, 63:64] - roy
    r2 = gx * gx + gy * gy
    inv = lax.rsqrt(r2)
    cc = jnp.where(r2 > 0.0, gx * inv, 1.0)
    ss = jnp.where(r2 > 0.0, gy * inv, 0.0)
    ax = _bfr(absx - rox)
    ay = _bfr(absy - roy)
    ccb = _bfr(cc)
    ssb = _bfr(ss)
    rrx = ax * ccb + ay * ssb
    rry = ay * ccb - ax * ssb
    recattr = jnp.concatenate([at[:, :10], sw], axis=1)
    mask60 = (recattr != 0.0).astype(_F32)
    rrx = jnp.concatenate([rrx[:, :60] * mask60, rrx[:, 60:]], axis=1)
    rry = jnp.concatenate([rry[:, :60] * mask60, rry[:, 60:]], axis=1)
    rrx_ref[...] = rrx
    rry_ref[...] = rry
    ro_ref[...] = jnp.concatenate([rox, roy], axis=1)
    rrot_ref[...] = jnp.concatenate([cc, -ss, ss, cc], axis=1)
    npx = np_ref[:, 0:1]
    npy = np_ref[:, 1:2]
    rnp_ref[...] = jnp.concatenate(
        [npx + rox - opx, npy + roy - opy], axis=1)


def _mid(rp, ox, oy, at, eps_s, noise, rotf, ori_pos, node_pos):
    return pl.pallas_call(
        _mid_body, grid=(GRID,),
        in_specs=[_row(10)] * 5 + [_row(64)] * 3 + [_row(10)] * 2
        + [_row(2), _row(4), _row(2), _row(2)],
        out_specs=[_row(64), _row(64), _row(2), _row(4), _row(2),
                   pl.BlockSpec((1, 1, HID), lambda i: (i, 0, 0))],
        out_shape=[jax.ShapeDtypeStruct((N, 64), _F32),
                   jax.ShapeDtypeStruct((N, 64), _F32),
                   jax.ShapeDtypeStruct((N, 2), _F32),
                   jax.ShapeDtypeStruct((N, 4), _F32),
                   jax.ShapeDtypeStruct((N, 2), _F32),
                   jax.ShapeDtypeStruct((GRID, 1, HID), _F32)],
    )(rp[..., 0], rp[..., 1], rp[..., 2], rp[..., 3], rp[..., 4],
      ox, oy, at, eps_s[..., 0], eps_s[..., 1], noise, rotf, ori_pos,
      node_pos)


def _graphnet_tail(h, y, e3, idx, p, n_layers, w_out_p, b_out_p):
    for l in range(n_layers):
        g3 = _gather_rows(y, idx).reshape(K, N, HID)
        if l == n_layers - 1:
            w_next, b_next = w_out_p, b_out_p
        else:
            w_next = p['W_msg%d' % (l + 1)][:HID, :]
            b_next = jnp.zeros((w_next.shape[1],), _F32)
        h, y = _layer(h, g3, e3, p['W_msg%d' % l][HID:, :], p['b_msg%d' % l],
                      p['W_upd%d' % l], p['b_upd%d' % l], w_next, b_next)
    return h, y


def _pad_out(w, b, dout):
    dpad = dout if dout % HID == 0 else (dout // HID + 1) * HID
    wp = jnp.zeros((HID, dpad), _F32).at[:, :dout].set(w)
    bp = jnp.zeros((dpad,), _F32).at[:dout].set(b)
    return wp, bp


def kernel(graph_state, node_pos, ori_pos, rotate, enc, dec, pol):
    gs3 = graph_state.reshape(N, 64, 3)
    ox = gs3[:, :, 0]
    oy = gs3[:, :, 1]
    at = gs3[:, :, 2]
    rotf = rotate.reshape(N, 4)
    eps = jax.random.normal(jax.random.key(11), (N, 64), _F32)
    eps_s = jax.random.normal(jax.random.key(12), (N, 10, 2), _F32)
    noise = 0.1 * jax.random.normal(jax.random.key(13), (N, 2), _F32)

    nbr, e3_enc, e3_dec = _make_knn(2)(
        node_pos, node_pos.T, rotf, enc['W_e'], enc['b_e'].reshape(1, HID),
        dec['W_e'], dec['b_e'].reshape(1, HID))
    idx = nbr.T.reshape(-1)

    h, y = _encin(graph_state, enc['W_in'], enc['b_in'],
                  enc['W_msg0'][:HID, :])
    _, y_enc = _graphnet_tail(h, y, e3_enc, idx, enc, 2,
                              enc['W_out'], enc['b_out'])

    wd = dec['W_in']
    wctx = wd[64:172].reshape(54, 2, HID)
    zpad = jnp.zeros((10, HID), _F32)
    wxd = jnp.concatenate([zpad, wctx[:, 0, :]], axis=0)
    wyd = jnp.concatenate([zpad, wctx[:, 1, :]], axis=0)
    h, y, klp = _decin(y_enc, eps, ox, oy, at, wd[:64], wxd, wyd,
                       wd[172:], dec['b_in'], dec['W_msg0'][:HID, :])
    wop, bop = _pad_out(dec['W_out'], dec['b_out'], 50)
    _, y_dec = _graphnet_tail(h, y, e3_dec, idx, dec, 2, wop, bop)
    kl = 0.5 * jnp.sum(klp[:, 0, 0]) / N

    rp = y_dec[:, :50].reshape(N, 10, 5)
    rrx, rry, ro, rrotf, rnp, lossp = _mid(rp, ox, oy, at, eps_s, noise,
                                           rotf, ori_pos, node_pos)
    rec_loss = jnp.sum(lossp[:, 0, 0]) / jnp.maximum(
        jnp.sum(lossp[:, 0, 1]), 1.0)

    nbr_p, e3_pol = _make_knn(1)(
        rnp, rnp.T, rrotf, pol['W_e'], pol['b_e'].reshape(1, HID))
    idx_p = nbr_p.T.reshape(-1)
    wp3 = pol['W_in'].reshape(64, 3, HID)
    h, y = _polin(rrx, rry, at, wp3[:, 0, :], wp3[:, 1, :], wp3[:, 2, :],
                  pol['b_in'], pol['W_msg0'][:HID, :])
    wopp, bopp = _pad_out(pol['W_out'], pol['b_out'], 250)
    _, y_pol = _graphnet_tail(h, y, e3_pol, idx_p, pol, 3, wopp, bopp)
    action_preds = y_pol[:, :250]

    rec_rot = rrotf.reshape(N, 2, 2)
    return action_preds, rec_loss, kl, ro, rec_rot


# speedup vs baseline: 9.4631x; 1.0512x over previous
import functools

import jax
import jax.numpy as jnp
from jax import lax
from jax.experimental import pallas as pl
from jax.experimental.pallas import tpu as pltpu
from jax.experimental.pallas import tpu_sc as plsc

N = 2048
HID = 128
K = 16
BLK = 256
GRID = N // BLK
LOG2PI = 1.8378770664093453


def _row(d):
    return pl.BlockSpec((BLK, d), lambda i: (i, 0))


def _full(*shape):
    return pl.BlockSpec(shape, lambda i: tuple(0 for _ in shape))


_KMAJ = pl.BlockSpec((K, BLK, HID), lambda i: (0, i, 0))
_F32 = jnp.float32


def _bdot(a, b):
    return jnp.dot(a.astype(jnp.bfloat16), b.astype(jnp.bfloat16),
                   preferred_element_type=_F32)


def _bfr(x):
    return x.astype(jnp.bfloat16).astype(_F32)


def _gather_rows(table, idx):
    mesh = plsc.VectorSubcoreMesh(core_axis_name="c", subcore_axis_name="s")
    n_out = idx.shape[0]
    per_w = n_out // 32
    n_chunk = per_w // 128

    @functools.partial(
        pl.kernel, mesh=mesh,
        out_type=jax.ShapeDtypeStruct((n_out, HID), _F32),
        scratch_types=[
            pltpu.VMEM((128,), jnp.int32),
            pltpu.VMEM((128,), jnp.int32),
            pltpu.VMEM((128, HID), _F32),
            pltpu.VMEM((128, HID), _F32),
            pltpu.SemaphoreType.DMA,
            pltpu.SemaphoreType.DMA,
            pltpu.SemaphoreType.DMA,
            pltpu.SemaphoreType.DMA,
        ],
    )
    def k(table_hbm, idx_hbm, out_hbm, idx0, idx1, rows0, rows1,
          gs0, gs1, ws0, ws1):
        wid = lax.axis_index("s") * 2 + lax.axis_index("c")
        base = wid * per_w
        idxv = (idx0, idx1)
        rows = (rows0, rows1)
        gsem = (gs0, gs1)
        wsem = (ws0, ws1)
        g = [None, None]
        wb = [None, None]
        pltpu.sync_copy(idx_hbm.at[pl.ds(base, 128)], idxv[0])
        g[0] = pltpu.async_copy(table_hbm.at[idxv[0]], rows[0], gsem[0])
        for c in range(n_chunk):
            b = c % 2
            nb = (c + 1) % 2
            if c + 1 < n_chunk:
                if c + 1 >= 2:
                    wb[nb].wait()
                pltpu.sync_copy(
                    idx_hbm.at[pl.ds(base + (c + 1) * 128, 128)], idxv[nb])
                g[nb] = pltpu.async_copy(
                    table_hbm.at[idxv[nb]], rows[nb], gsem[nb])
            g[b].wait()
            wb[b] = pltpu.async_copy(
                rows[b], out_hbm.at[pl.ds(base + c * 128, 128)], wsem[b])
        wb[(n_chunk - 2) % 2].wait()
        wb[(n_chunk - 1) % 2].wait()

    return k(table, idx)


def _knn_body(pos_ref, posT_ref, rot_ref, nbr_ref, relx_ref, rely_ref):
    px = pos_ref[:, 0:1]
    py = pos_ref[:, 1:2]
    qx = posT_ref[0:1, :]
    qy = posT_ref[1:2, :]
    dx = qx - px
    dy = qy - py
    dist = jnp.sqrt(dx * dx + dy * dy + 1e-9)
    jj = lax.broadcasted_iota(jnp.int32, dist.shape, 1)
    r00b = _bfr(rot_ref[:, 0:1])
    r01b = _bfr(rot_ref[:, 1:2])
    r10b = _bfr(rot_ref[:, 2:3])
    r11b = _bfr(rot_ref[:, 3:4])
    idx_cols = []
    for k in range(K):
        m = jnp.min(dist, axis=1, keepdims=True)
        idx = jnp.min(jnp.where(dist == m, jj, N), axis=1, keepdims=True)
        sel = jj == idx
        dxk = jnp.sum(jnp.where(sel, dx, 0.0), axis=1, keepdims=True)
        dyk = jnp.sum(jnp.where(sel, dy, 0.0), axis=1, keepdims=True)
        dist = jnp.where(sel, jnp.inf, dist)
        idx_cols.append(idx)
        dxb = _bfr(dxk)
        dyb = _bfr(dyk)
        relx_ref[:, k:k + 1] = _bfr(dxb * r00b + dyb * r10b)
        rely_ref[:, k:k + 1] = _bfr(dxb * r01b + dyb * r11b)
    nbr_ref[...] = jnp.concatenate(idx_cols, axis=1)


def _knn(pos, rotf):
    return pl.pallas_call(
        _knn_body, grid=(GRID,),
        in_specs=[_row(2), _full(2, N), _row(4)],
        out_specs=[pl.BlockSpec((BLK, K), lambda i: (i, 0)),
                   _row(K), _row(K)],
        out_shape=[jax.ShapeDtypeStruct((N, K), jnp.int32),
                   jax.ShapeDtypeStruct((N, K), _F32),
                   jax.ShapeDtypeStruct((N, K), _F32)],
    )(pos, pos.T, rotf)


def _encin_body(x_ref, wi_ref, bi_ref, wm_ref, h_ref, y_ref):
    h = jnp.maximum(
        _bdot(x_ref[...], wi_ref[...])
        + bi_ref[...], 0.0)
    h_ref[...] = h
    y_ref[...] = _bdot(h, wm_ref[...])


def _encin(x, wi, bi, wm):
    din = x.shape[1]
    return pl.pallas_call(
        _encin_body, grid=(GRID,),
        in_specs=[_row(din), _full(din, HID), _full(1, HID),
                  _full(HID, HID)],
        out_specs=[_row(HID), _row(HID)],
        out_shape=[jax.ShapeDtypeStruct((N, HID), _F32),
                   jax.ShapeDtypeStruct((N, HID), _F32)],
    )(x, wi, bi.reshape(1, HID), wm)


def _layer_body(h_ref, g3_ref, relx_ref, rely_ref, we_ref, be_ref,
                wb_ref, bm_ref, wu_ref, bu_ref,
                wn_ref, bn_ref, h_out_ref, y_ref):
    we0b = _bfr(we_ref[0:1, :])
    we1b = _bfr(we_ref[1:2, :])
    be = be_ref[...]
    e_ks = []
    for k in range(K):
        rx = relx_ref[:, k:k + 1]
        ry = rely_ref[:, k:k + 1]
        e_ks.append(jnp.maximum(rx * we0b + ry * we1b + be, 0.0))
    e = jnp.concatenate(e_ks, axis=0)
    g = g3_ref[...].reshape(K * BLK, HID)
    eb = _bdot(e, wb_ref[...])
    msg = jnp.maximum(g + eb + bm_ref[...], 0.0)
    m3 = msg.reshape(K, BLK, HID)
    acc = m3[0]
    for k in range(1, K):
        acc = acc + m3[k]
    agg = acc * (1.0 / K)
    h = h_ref[...]
    u = jnp.maximum(
        _bdot(h, wu_ref[:HID, :])
        + _bdot(agg, wu_ref[HID:, :])
        + bu_ref[...], 0.0)
    hn = h + u
    h_out_ref[...] = hn
    y_ref[...] = _bdot(hn, wn_ref[...]) \
        + bn_ref[...]


def _layer(h, g3, relx, rely, we, be, w_msg_bot, b_msg, w_upd, b_upd,
           w_next, b_next):
    dout = w_next.shape[1]
    return pl.pallas_call(
        _layer_body, grid=(GRID,),
        in_specs=[_row(HID), _KMAJ, _row(K), _row(K), _full(2, HID),
                  _full(1, HID), _full(HID, HID), _full(1, HID),
                  _full(2 * HID, HID), _full(1, HID), _full(HID, dout),
                  _full(1, dout)],
        out_specs=[_row(HID), _row(dout)],
        out_shape=[jax.ShapeDtypeStruct((N, HID), _F32),
                   jax.ShapeDtypeStruct((N, dout), _F32)],
    )(h, g3, relx, rely, we, be.reshape(1, HID), w_msg_bot,
      b_msg.reshape(1, HID), w_upd, b_upd.reshape(1, HID), w_next,
      b_next.reshape(1, dout))


def _decin_body(he_ref, eps_ref, ox_ref, oy_ref, at_ref, wz_ref, wx_ref,
                wy_ref, wa_ref, b_ref, wm_ref, h_ref, y_ref, klp_ref):
    he = he_ref[...]
    mu = he[:, :64]
    lv = he[:, 64:]
    elv = jnp.exp(lv)
    z = mu + eps_ref[...] * jnp.exp(lv * 0.5)
    klv = jnp.sum(elv + mu * mu - lv - 1.0)
    klp_ref[...] = jnp.broadcast_to(klv.reshape(1, 1, 1), (1, 1, HID))
    h0 = jnp.maximum(
        _bdot(z, wz_ref[...])
        + _bdot(ox_ref[...], wx_ref[...])
        + _bdot(oy_ref[...], wy_ref[...])
        + _bdot(at_ref[...], wa_ref[...])
        + b_ref[...], 0.0)
    h_ref[...] = h0
    y_ref[...] = _bdot(h0, wm_ref[...])


def _decin(he, eps, ox, oy, at, wz, wx, wy, wa, b, wm):
    return pl.pallas_call(
        _decin_body, grid=(GRID,),
        in_specs=[_row(HID), _row(64), _row(64), _row(64), _row(64),
                  _full(64, HID), _full(64, HID), _full(64, HID),
                  _full(64, HID), _full(1, HID), _full(HID, HID)],
        out_specs=[_row(HID), _row(HID),
                   pl.BlockSpec((1, 1, HID), lambda i: (i, 0, 0))],
        out_shape=[jax.ShapeDtypeStruct((N, HID), _F32),
                   jax.ShapeDtypeStruct((N, HID), _F32),
                   jax.ShapeDtypeStruct((GRID, 1, HID), _F32)],
    )(he, eps, ox, oy, at, wz, wx, wy, wa, b.reshape(1, HID), wm)


def _polin_body(rx_ref, ry_ref, at_ref, wx_ref, wy_ref, wa_ref, b_ref,
                wm_ref, h_ref, y_ref):
    h0 = jnp.maximum(
        _bdot(rx_ref[...], wx_ref[...])
        + _bdot(ry_ref[...], wy_ref[...])
        + _bdot(at_ref[...], wa_ref[...])
        + b_ref[...], 0.0)
    h_ref[...] = h0
    y_ref[...] = _bdot(h0, wm_ref[...])


def _polin(rx, ry, at, wx, wy, wa, b, wm):
    return pl.pallas_call(
        _polin_body, grid=(GRID,),
        in_specs=[_row(64), _row(64), _row(64), _full(64, HID),
                  _full(64, HID), _full(64, HID), _full(1, HID),
                  _full(HID, HID)],
        out_specs=[_row(HID), _row(HID)],
        out_shape=[jax.ShapeDtypeStruct((N, HID), _F32),
                   jax.ShapeDtypeStruct((N, HID), _F32)],
    )(rx, ry, at, wx, wy, wa, b.reshape(1, HID), wm)


def _mid_body(rp0_ref, rp1_ref, rp2_ref, rp3_ref, rp4_ref, ox_ref, oy_ref,
              at_ref, e0_ref, e1_ref, nz_ref, rot_ref, op_ref, np_ref,
              rrx_ref, rry_ref, ro_ref, rrot_ref, rnp_ref, lp_ref):
    ox = ox_ref[...]
    oy = oy_ref[...]
    at = at_ref[...]
    mx = rp0_ref[...]
    my = rp1_ref[...]
    off = rp2_ref[...]
    lp3 = rp3_ref[...]
    lp4 = rp4_ref[...]
    d0 = jnp.exp(lp3)
    d1 = jnp.exp(lp4)
    y0 = (ox[:, :10] - mx) / d0
    y1 = (oy[:, :10] - my - off * y0) / d1
    neg_logp = 0.5 * (y0 * y0 + y1 * y1) + lp3 + lp4 + LOG2PI
    mask10 = (at[:, :10] != 0.0).astype(_F32)
    nl_sum = jnp.sum(neg_logp * mask10).reshape(1, 1)
    m_sum = jnp.sum(mask10).reshape(1, 1)
    lp_ref[...] = jnp.concatenate(
        [nl_sum, m_sum, jnp.zeros((1, HID - 2), _F32)], axis=1
    ).reshape(1, 1, HID)
    e0 = e0_ref[...]
    e1 = e1_ref[...]
    s0 = (mx + d0 * e0) * mask10
    s1 = (my + off * e0 + d1 * e1) * mask10
    rsx = jnp.concatenate([s0, ox[:, 10:]], axis=1)
    rsy = jnp.concatenate([s1, oy[:, 10:]], axis=1)
    nx = nz_ref[:, 0:1]
    ny = nz_ref[:, 1:2]
    curx = rsx[:, 0:1] + nx
    cury = rsy[:, 0:1] + ny
    rpx = ox[:, 10:60]
    rpy = oy[:, 10:60]
    wdt = at[:, 10:60]
    ddx = rpx - curx
    ddy = rpy - cury
    avail = (wdt != 0.0).astype(_F32)
    dd = jnp.sqrt(ddx * ddx + ddy * ddy) - wdt - avail * 1000.0
    jj = lax.broadcasted_iota(jnp.int32, dd.shape, 1)
    sx = jnp.zeros(dd.shape, _F32)
    sy = jnp.zeros(dd.shape, _F32)
    sw = jnp.zeros(dd.shape, _F32)
    for i in range(50):
        di = dd[:, i:i + 1]
        less = (dd < di).astype(jnp.int32)
        eq = ((dd == di) & (jj < i)).astype(jnp.int32)
        rank = jnp.sum(less + eq, axis=1, keepdims=True)
        oh = (jj == rank).astype(_F32)
        sx = sx + oh * rpx[:, i:i + 1]
        sy = sy + oh * rpy[:, i:i + 1]
        sw = sw + oh * wdt[:, i:i + 1]
    rsx = jnp.concatenate([rsx[:, :10], sx, rsx[:, 60:]], axis=1)
    rsy = jnp.concatenate([rsy[:, :10], sy, rsy[:, 60:]], axis=1)
    c0 = rot_ref[:, 0:1]
    c1 = rot_ref[:, 1:2]
    c2 = rot_ref[:, 2:3]
    c3 = rot_ref[:, 3:4]
    opx = op_ref[:, 0:1]
    opy = op_ref[:, 1:2]
    rsxb = _bfr(rsx)
    rsyb = _bfr(rsy)
    c0b = _bfr(c0)
    c1b = _bfr(c1)
    c2b = _bfr(c2)
    c3b = _bfr(c3)
    absx = rsxb * c0b + rsyb * c1b + opx
    absy = rsxb * c2b + rsyb * c3b + opy
    nrx = _bfr(nx) * c0b + _bfr(ny) * c1b
    nry = _bfr(nx) * c2b + _bfr(ny) * c3b
    rox = absx[:, 0:1] + nrx
    roy = absy[:, 0:1] + nry
    gx = absx[:, 63:64] - rox
    gy = absy[:, 63:64] - roy
    r2 = gx * gx + gy * gy
    inv = lax.rsqrt(r2)
    cc = jnp.where(r2 > 0.0, gx * inv, 1.0)
    ss = jnp.where(r2 > 0.0, gy * inv, 0.0)
    ax = _bfr(absx - rox)
    ay = _bfr(absy - roy)
    ccb = _bfr(cc)
    ssb = _bfr(ss)
    rrx = ax * ccb + ay * ssb
    rry = ay * ccb - ax * ssb
    recattr = jnp.concatenate([at[:, :10], sw], axis=1)
    mask60 = (recattr != 0.0).astype(_F32)
    rrx = jnp.concatenate([rrx[:, :60] * mask60, rrx[:, 60:]], axis=1)
    rry = jnp.concatenate([rry[:, :60] * mask60, rry[:, 60:]], axis=1)
    rrx_ref[...] = rrx
    rry_ref[...] = rry
    ro_ref[...] = jnp.concatenate([rox, roy], axis=1)
    rrot_ref[...] = jnp.concatenate([cc, -ss, ss, cc], axis=1)
    npx = np_ref[:, 0:1]
    npy = np_ref[:, 1:2]
    rnp_ref[...] = jnp.concatenate(
        [npx + rox - opx, npy + roy - opy], axis=1)


def _mid(rp, ox, oy, at, eps_s, noise, rotf, ori_pos, node_pos):
    return pl.pallas_call(
        _mid_body, grid=(GRID,),
        in_specs=[_row(10)] * 5 + [_row(64)] * 3 + [_row(10)] * 2
        + [_row(2), _row(4), _row(2), _row(2)],
        out_specs=[_row(64), _row(64), _row(2), _row(4), _row(2),
                   pl.BlockSpec((1, 1, HID), lambda i: (i, 0, 0))],
        out_shape=[jax.ShapeDtypeStruct((N, 64), _F32),
                   jax.ShapeDtypeStruct((N, 64), _F32),
                   jax.ShapeDtypeStruct((N, 2), _F32),
                   jax.ShapeDtypeStruct((N, 4), _F32),
                   jax.ShapeDtypeStruct((N, 2), _F32),
                   jax.ShapeDtypeStruct((GRID, 1, HID), _F32)],
    )(rp[..., 0], rp[..., 1], rp[..., 2], rp[..., 3], rp[..., 4],
      ox, oy, at, eps_s[..., 0], eps_s[..., 1], noise, rotf, ori_pos,
      node_pos)


def _graphnet_tail(h, y, relx, rely, idx, p, n_layers, w_out_p, b_out_p):
    for l in range(n_layers):
        g3 = _gather_rows(y, idx).reshape(K, N, HID)
        if l == n_layers - 1:
            w_next, b_next = w_out_p, b_out_p
        else:
            w_next = p['W_msg%d' % (l + 1)][:HID, :]
            b_next = jnp.zeros((w_next.shape[1],), _F32)
        h, y = _layer(h, g3, relx, rely, p['W_e'], p['b_e'],
                      p['W_msg%d' % l][HID:, :], p['b_msg%d' % l],
                      p['W_upd%d' % l], p['b_upd%d' % l], w_next, b_next)
    return h, y


def _pad_out(w, b, dout):
    dpad = dout if dout % HID == 0 else (dout // HID + 1) * HID
    wp = jnp.zeros((HID, dpad), _F32).at[:, :dout].set(w)
    bp = jnp.zeros((dpad,), _F32).at[:dout].set(b)
    return wp, bp


def kernel(graph_state, node_pos, ori_pos, rotate, enc, dec, pol):
    gs3 = graph_state.reshape(N, 64, 3)
    ox = gs3[:, :, 0]
    oy = gs3[:, :, 1]
    at = gs3[:, :, 2]
    rotf = rotate.reshape(N, 4)
    eps = jax.random.normal(jax.random.key(11), (N, 64), _F32)
    eps_s = jax.random.normal(jax.random.key(12), (N, 10, 2), _F32)
    noise = 0.1 * jax.random.normal(jax.random.key(13), (N, 2), _F32)

    nbr, relx, rely = _knn(node_pos, rotf)
    idx = nbr.T.reshape(-1)

    h, y = _encin(graph_state, enc['W_in'], enc['b_in'],
                  enc['W_msg0'][:HID, :])
    _, y_enc = _graphnet_tail(h, y, relx, rely, idx, enc, 2,
                              enc['W_out'], enc['b_out'])

    wd = dec['W_in']
    wctx = wd[64:172].reshape(54, 2, HID)
    zpad = jnp.zeros((10, HID), _F32)
    wxd = jnp.concatenate([zpad, wctx[:, 0, :]], axis=0)
    wyd = jnp.concatenate([zpad, wctx[:, 1, :]], axis=0)
    h, y, klp = _decin(y_enc, eps, ox, oy, at, wd[:64], wxd, wyd,
                       wd[172:], dec['b_in'], dec['W_msg0'][:HID, :])
    wop, bop = _pad_out(dec['W_out'], dec['b_out'], 50)
    _, y_dec = _graphnet_tail(h, y, relx, rely, idx, dec, 2, wop, bop)
    kl = 0.5 * jnp.sum(klp[:, 0, 0]) / N

    rp = y_dec[:, :50].reshape(N, 10, 5)
    rrx, rry, ro, rrotf, rnp, lossp = _mid(rp, ox, oy, at, eps_s, noise,
                                           rotf, ori_pos, node_pos)
    rec_loss = jnp.sum(lossp[:, 0, 0]) / jnp.maximum(
        jnp.sum(lossp[:, 0, 1]), 1.0)

    nbr_p, relx_p, rely_p = _knn(rnp, rrotf)
    idx_p = nbr_p.T.reshape(-1)
    wp3 = pol['W_in'].reshape(64, 3, HID)
    h, y = _polin(rrx, rry, at, wp3[:, 0, :], wp3[:, 1, :], wp3[:, 2, :],
                  pol['b_in'], pol['W_msg0'][:HID, :])
    wopp, bopp = _pad_out(pol['W_out'], pol['b_out'], 250)
    _, y_pol = _graphnet_tail(h, y, relx_p, rely_p, idx_p, pol, 3,
                              wopp, bopp)
    action_preds = y_pol[:, :250]

    rec_rot = rrotf.reshape(N, 2, 2)
    return action_preds, rec_loss, kl, ro, rec_rot


# speedup vs baseline: 10.8133x; 1.1427x over previous
import functools

import jax
import jax.numpy as jnp
from jax import lax
from jax.experimental import pallas as pl
from jax.experimental.pallas import tpu as pltpu
from jax.experimental.pallas import tpu_sc as plsc

N = 2048
HID = 128
K = 16
BLK = 256
GRID = N // BLK
LOG2PI = 1.8378770664093453


def _row(d):
    return pl.BlockSpec((BLK, d), lambda i: (i, 0))


def _full(*shape):
    return pl.BlockSpec(shape, lambda i: tuple(0 for _ in shape))


_KMAJ = pl.BlockSpec((K, BLK, HID), lambda i: (0, i, 0))
_F32 = jnp.float32


def _bdot(a, b):
    return jnp.dot(a.astype(jnp.bfloat16), b.astype(jnp.bfloat16),
                   preferred_element_type=_F32)


def _bfr(x):
    return x.astype(jnp.bfloat16).astype(_F32)


def _gather_rows(table, idx):
    mesh = plsc.VectorSubcoreMesh(core_axis_name="c", subcore_axis_name="s")
    n_out = idx.shape[0]
    width = table.shape[1]
    per_w = n_out // 32
    n_chunk = per_w // 128

    @functools.partial(
        pl.kernel, mesh=mesh,
        out_type=jax.ShapeDtypeStruct((n_out, width), _F32),
        scratch_types=[
            pltpu.VMEM((128,), jnp.int32),
            pltpu.VMEM((128,), jnp.int32),
            pltpu.VMEM((128, width), _F32),
            pltpu.VMEM((128, width), _F32),
            pltpu.SemaphoreType.DMA,
            pltpu.SemaphoreType.DMA,
            pltpu.SemaphoreType.DMA,
            pltpu.SemaphoreType.DMA,
        ],
    )
    def k(table_hbm, idx_hbm, out_hbm, idx0, idx1, rows0, rows1,
          gs0, gs1, ws0, ws1):
        wid = lax.axis_index("s") * 2 + lax.axis_index("c")
        base = wid * per_w
        idxv = (idx0, idx1)
        rows = (rows0, rows1)
        gsem = (gs0, gs1)
        wsem = (ws0, ws1)
        g = [None, None]
        wb = [None, None]
        pltpu.sync_copy(idx_hbm.at[pl.ds(base, 128)], idxv[0])
        g[0] = pltpu.async_copy(table_hbm.at[idxv[0]], rows[0], gsem[0])
        for c in range(n_chunk):
            b = c % 2
            nb = (c + 1) % 2
            if c + 1 < n_chunk:
                if c + 1 >= 2:
                    wb[nb].wait()
                pltpu.sync_copy(
                    idx_hbm.at[pl.ds(base + (c + 1) * 128, 128)], idxv[nb])
                g[nb] = pltpu.async_copy(
                    table_hbm.at[idxv[nb]], rows[nb], gsem[nb])
            g[b].wait()
            wb[b] = pltpu.async_copy(
                rows[b], out_hbm.at[pl.ds(base + c * 128, 128)], wsem[b])
        wb[(n_chunk - 2) % 2].wait()
        wb[(n_chunk - 1) % 2].wait()

    return k(table, idx)


def _knn_body(pos_ref, posT_ref, nbr_ref):
    px = pos_ref[:, 0:1]
    py = pos_ref[:, 1:2]
    qx = posT_ref[0:1, :]
    qy = posT_ref[1:2, :]
    dx = qx - px
    dy = qy - py
    dist = jnp.sqrt(dx * dx + dy * dy + 1e-9)
    jj = lax.broadcasted_iota(jnp.int32, dist.shape, 1)
    idx_cols = []
    for k in range(K):
        m = jnp.min(dist, axis=1, keepdims=True)
        idx = jnp.min(jnp.where(dist == m, jj, N), axis=1, keepdims=True)
        idx_cols.append(idx)
        if k < K - 1:
            dist = jnp.where(jj == idx, jnp.inf, dist)
    nbr_ref[...] = jnp.concatenate(idx_cols, axis=1)


def _knn(pos):
    return pl.pallas_call(
        _knn_body, grid=(GRID,),
        in_specs=[_row(2), _full(2, N)],
        out_specs=pl.BlockSpec((BLK, K), lambda i: (i, 0)),
        out_shape=jax.ShapeDtypeStruct((N, K), jnp.int32),
    )(pos, pos.T)


def _edges_body(pg_ref, pos_ref, rot_ref, relx_ref, rely_ref):
    px = pos_ref[:, 0:1]
    py = pos_ref[:, 1:2]
    r00b = _bfr(rot_ref[:, 0:1])
    r01b = _bfr(rot_ref[:, 1:2])
    r10b = _bfr(rot_ref[:, 2:3])
    r11b = _bfr(rot_ref[:, 3:4])
    for k in range(K):
        dxb = _bfr(pg_ref[k, :, 0:1] - px)
        dyb = _bfr(pg_ref[k, :, 1:2] - py)
        relx_ref[:, k:k + 1] = _bfr(dxb * r00b + dyb * r10b)
        rely_ref[:, k:k + 1] = _bfr(dxb * r01b + dyb * r11b)


def _edges(pg3, pos, rotf):
    return pl.pallas_call(
        _edges_body, grid=(GRID,),
        in_specs=[pl.BlockSpec((K, BLK, HID), lambda i: (0, i, 0)),
                  _row(2), _row(4)],
        out_specs=[_row(K), _row(K)],
        out_shape=[jax.ShapeDtypeStruct((N, K), _F32),
                   jax.ShapeDtypeStruct((N, K), _F32)],
    )(pg3, pos, rotf)


def _encin_body(x_ref, wi_ref, bi_ref, wm_ref, h_ref, y_ref):
    h = jnp.maximum(
        _bdot(x_ref[...], wi_ref[...])
        + bi_ref[...], 0.0)
    h_ref[...] = h
    y_ref[...] = _bdot(h, wm_ref[...])


def _encin(x, wi, bi, wm):
    din = x.shape[1]
    return pl.pallas_call(
        _encin_body, grid=(GRID,),
        in_specs=[_row(din), _full(din, HID), _full(1, HID),
                  _full(HID, HID)],
        out_specs=[_row(HID), _row(HID)],
        out_shape=[jax.ShapeDtypeStruct((N, HID), _F32),
                   jax.ShapeDtypeStruct((N, HID), _F32)],
    )(x, wi, bi.reshape(1, HID), wm)


def _layer_body(h_ref, g3_ref, relx_ref, rely_ref, we_ref, be_ref,
                wb_ref, bm_ref, wu_ref, bu_ref,
                wn_ref, bn_ref, h_out_ref, y_ref):
    we0b = _bfr(we_ref[0:1, :])
    we1b = _bfr(we_ref[1:2, :])
    be = be_ref[...]
    e_ks = []
    for k in range(K):
        rx = relx_ref[:, k:k + 1]
        ry = rely_ref[:, k:k + 1]
        e_ks.append(jnp.maximum(rx * we0b + ry * we1b + be, 0.0))
    e = jnp.concatenate(e_ks, axis=0)
    g = g3_ref[...].reshape(K * BLK, HID)
    eb = _bdot(e, wb_ref[...])
    msg = jnp.maximum(g + eb + bm_ref[...], 0.0)
    m3 = msg.reshape(K, BLK, HID)
    acc = m3[0]
    for k in range(1, K):
        acc = acc + m3[k]
    agg = acc * (1.0 / K)
    h = h_ref[...]
    u = jnp.maximum(
        _bdot(h, wu_ref[:HID, :])
        + _bdot(agg, wu_ref[HID:, :])
        + bu_ref[...], 0.0)
    hn = h + u
    h_out_ref[...] = hn
    y_ref[...] = _bdot(hn, wn_ref[...]) \
        + bn_ref[...]


def _layer(h, g3, relx, rely, we, be, w_msg_bot, b_msg, w_upd, b_upd,
           w_next, b_next):
    dout = w_next.shape[1]
    return pl.pallas_call(
        _layer_body, grid=(GRID,),
        in_specs=[_row(HID), _KMAJ, _row(K), _row(K), _full(2, HID),
                  _full(1, HID), _full(HID, HID), _full(1, HID),
                  _full(2 * HID, HID), _full(1, HID), _full(HID, dout),
                  _full(1, dout)],
        out_specs=[_row(HID), _row(dout)],
        out_shape=[jax.ShapeDtypeStruct((N, HID), _F32),
                   jax.ShapeDtypeStruct((N, dout), _F32)],
    )(h, g3, relx, rely, we, be.reshape(1, HID), w_msg_bot,
      b_msg.reshape(1, HID), w_upd, b_upd.reshape(1, HID), w_next,
      b_next.reshape(1, dout))


def _decin_body(he_ref, eps_ref, ox_ref, oy_ref, at_ref, wz_ref, wx_ref,
                wy_ref, wa_ref, b_ref, wm_ref, h_ref, y_ref, klp_ref):
    he = he_ref[...]
    mu = he[:, :64]
    lv = he[:, 64:]
    elv = jnp.exp(lv)
    z = mu + eps_ref[...] * jnp.exp(lv * 0.5)
    klv = jnp.sum(elv + mu * mu - lv - 1.0)
    klp_ref[...] = jnp.broadcast_to(klv.reshape(1, 1, 1), (1, 1, HID))
    h0 = jnp.maximum(
        _bdot(z, wz_ref[...])
        + _bdot(ox_ref[...], wx_ref[...])
        + _bdot(oy_ref[...], wy_ref[...])
        + _bdot(at_ref[...], wa_ref[...])
        + b_ref[...], 0.0)
    h_ref[...] = h0
    y_ref[...] = _bdot(h0, wm_ref[...])


def _decin(he, eps, ox, oy, at, wz, wx, wy, wa, b, wm):
    return pl.pallas_call(
        _decin_body, grid=(GRID,),
        in_specs=[_row(HID), _row(64), _row(64), _row(64), _row(64),
                  _full(64, HID), _full(64, HID), _full(64, HID),
                  _full(64, HID), _full(1, HID), _full(HID, HID)],
        out_specs=[_row(HID), _row(HID),
                   pl.BlockSpec((1, 1, HID), lambda i: (i, 0, 0))],
        out_shape=[jax.ShapeDtypeStruct((N, HID), _F32),
                   jax.ShapeDtypeStruct((N, HID), _F32),
                   jax.ShapeDtypeStruct((GRID, 1, HID), _F32)],
    )(he, eps, ox, oy, at, wz, wx, wy, wa, b.reshape(1, HID), wm)


def _polin_body(rx_ref, ry_ref, at_ref, wx_ref, wy_ref, wa_ref, b_ref,
                wm_ref, h_ref, y_ref):
    h0 = jnp.maximum(
        _bdot(rx_ref[...], wx_ref[...])
        + _bdot(ry_ref[...], wy_ref[...])
        + _bdot(at_ref[...], wa_ref[...])
        + b_ref[...], 0.0)
    h_ref[...] = h0
    y_ref[...] = _bdot(h0, wm_ref[...])


def _polin(rx, ry, at, wx, wy, wa, b, wm):
    return pl.pallas_call(
        _polin_body, grid=(GRID,),
        in_specs=[_row(64), _row(64), _row(64), _full(64, HID),
                  _full(64, HID), _full(64, HID), _full(1, HID),
                  _full(HID, HID)],
        out_specs=[_row(HID), _row(HID)],
        out_shape=[jax.ShapeDtypeStruct((N, HID), _F32),
                   jax.ShapeDtypeStruct((N, HID), _F32)],
    )(rx, ry, at, wx, wy, wa, b.reshape(1, HID), wm)


def _mid_body(rp0_ref, rp1_ref, rp2_ref, rp3_ref, rp4_ref, ox_ref, oy_ref,
              at_ref, e0_ref, e1_ref, nz_ref, rot_ref, op_ref, np_ref,
              rrx_ref, rry_ref, ro_ref, rrot_ref, rnp_ref, lp_ref):
    ox = ox_ref[...]
    oy = oy_ref[...]
    at = at_ref[...]
    mx = rp0_ref[...]
    my = rp1_ref[...]
    off = rp2_ref[...]
    lp3 = rp3_ref[...]
    lp4 = rp4_ref[...]
    d0 = jnp.exp(lp3)
    d1 = jnp.exp(lp4)
    y0 = (ox[:, :10] - mx) / d0
    y1 = (oy[:, :10] - my - off * y0) / d1
    neg_logp = 0.5 * (y0 * y0 + y1 * y1) + lp3 + lp4 + LOG2PI
    mask10 = (at[:, :10] != 0.0).astype(_F32)
    nl_sum = jnp.sum(neg_logp * mask10).reshape(1, 1)
    m_sum = jnp.sum(mask10).reshape(1, 1)
    lp_ref[...] = jnp.concatenate(
        [nl_sum, m_sum, jnp.zeros((1, HID - 2), _F32)], axis=1
    ).reshape(1, 1, HID)
    e0 = e0_ref[...]
    e1 = e1_ref[...]
    s0 = (mx + d0 * e0) * mask10
    s1 = (my + off * e0 + d1 * e1) * mask10
    rsx = jnp.concatenate([s0, ox[:, 10:]], axis=1)
    rsy = jnp.concatenate([s1, oy[:, 10:]], axis=1)
    nx = nz_ref[:, 0:1]
    ny = nz_ref[:, 1:2]
    curx = rsx[:, 0:1] + nx
    cury = rsy[:, 0:1] + ny
    rpx = ox[:, 10:60]
    rpy = oy[:, 10:60]
    wdt = at[:, 10:60]
    ddx = rpx - curx
    ddy = rpy - cury
    avail = (wdt != 0.0).astype(_F32)
    dd = jnp.sqrt(ddx * ddx + ddy * ddy) - wdt - avail * 1000.0
    dda = dd.reshape(BLK, 50, 1)
    ddb = dd.reshape(BLK, 1, 50)
    iii = lax.broadcasted_iota(jnp.int32, (BLK, 50, 50), 1)
    jjj = lax.broadcasted_iota(jnp.int32, (BLK, 50, 50), 2)
    before = (ddb < dda) | ((ddb == dda) & (jjj < iii))
    rank = jnp.sum(before.astype(jnp.int32), axis=2)
    oh = (rank.reshape(BLK, 50, 1) == jjj).astype(_F32)
    sx = jnp.sum(oh * rpx.reshape(BLK, 50, 1), axis=1)
    sy = jnp.sum(oh * rpy.reshape(BLK, 50, 1), axis=1)
    sw = jnp.sum(oh * wdt.reshape(BLK, 50, 1), axis=1)
    rsx = jnp.concatenate([rsx[:, :10], sx, rsx[:, 60:]], axis=1)
    rsy = jnp.concatenate([rsy[:, :10], sy, rsy[:, 60:]], axis=1)
    c0 = rot_ref[:, 0:1]
    c1 = rot_ref[:, 1:2]
    c2 = rot_ref[:, 2:3]
    c3 = rot_ref[:, 3:4]
    opx = op_ref[:, 0:1]
    opy = op_ref[:, 1:2]
    rsxb = _bfr(rsx)
    rsyb = _bfr(rsy)
    c0b = _bfr(c0)
    c1b = _bfr(c1)
    c2b = _bfr(c2)
    c3b = _bfr(c3)
    absx = rsxb * c0b + rsyb * c1b + opx
    absy = rsxb * c2b + rsyb * c3b + opy
    nrx = _bfr(nx) * c0b + _bfr(ny) * c1b
    nry = _bfr(nx) * c2b + _bfr(ny) * c3b
    rox = absx[:, 0:1] + nrx
    roy = absy[:, 0:1] + nry
    gx = absx[:, 63:64] - rox
    gy = absy[:, 63:64] - roy
    r2 = gx * gx + gy * gy
    inv = lax.rsqrt(r2)
    cc = jnp.where(r2 > 0.0, gx * inv, 1.0)
    ss = jnp.where(r2 > 0.0, gy * inv, 0.0)
    ax = _bfr(absx - rox)
    ay = _bfr(absy - roy)
    ccb = _bfr(cc)
    ssb = _bfr(ss)
    rrx = ax * ccb + ay * ssb
    rry = ay * ccb - ax * ssb
    recattr = jnp.concatenate([at[:, :10], sw], axis=1)
    mask60 = (recattr != 0.0).astype(_F32)
    rrx = jnp.concatenate([rrx[:, :60] * mask60, rrx[:, 60:]], axis=1)
    rry = jnp.concatenate([rry[:, :60] * mask60, rry[:, 60:]], axis=1)
    rrx_ref[...] = rrx
    rry_ref[...] = rry
    ro_ref[...] = jnp.concatenate([rox, roy], axis=1)
    rrot_ref[...] = jnp.concatenate([cc, -ss, ss, cc], axis=1)
    npx = np_ref[:, 0:1]
    npy = np_ref[:, 1:2]
    rnp_ref[...] = jnp.concatenate(
        [npx + rox - opx, npy + roy - opy], axis=1)


def _mid(rp, ox, oy, at, eps_s, noise, rotf, ori_pos, node_pos):
    return pl.pallas_call(
        _mid_body, grid=(GRID,),
        in_specs=[_row(10)] * 5 + [_row(64)] * 3 + [_row(10)] * 2
        + [_row(2), _row(4), _row(2), _row(2)],
        out_specs=[_row(64), _row(64), _row(2), _row(4), _row(2),
                   pl.BlockSpec((1, 1, HID), lambda i: (i, 0, 0))],
        out_shape=[jax.ShapeDtypeStruct((N, 64), _F32),
                   jax.ShapeDtypeStruct((N, 64), _F32),
                   jax.ShapeDtypeStruct((N, 2), _F32),
                   jax.ShapeDtypeStruct((N, 4), _F32),
                   jax.ShapeDtypeStruct((N, 2), _F32),
                   jax.ShapeDtypeStruct((GRID, 1, HID), _F32)],
    )(rp[..., 0], rp[..., 1], rp[..., 2], rp[..., 3], rp[..., 4],
      ox, oy, at, eps_s[..., 0], eps_s[..., 1], noise, rotf, ori_pos,
      node_pos)


def _graphnet_tail(h, y, relx, rely, idx, p, n_layers, w_out_p, b_out_p):
    for l in range(n_layers):
        g3 = _gather_rows(y, idx).reshape(K, N, HID)
        if l == n_layers - 1:
            w_next, b_next = w_out_p, b_out_p
        else:
            w_next = p['W_msg%d' % (l + 1)][:HID, :]
            b_next = jnp.zeros((w_next.shape[1],), _F32)
        h, y = _layer(h, g3, relx, rely, p['W_e'], p['b_e'],
                      p['W_msg%d' % l][HID:, :], p['b_msg%d' % l],
                      p['W_upd%d' % l], p['b_upd%d' % l], w_next, b_next)
    return h, y


def _pad_out(w, b, dout):
    dpad = dout if dout % HID == 0 else (dout // HID + 1) * HID
    wp = jnp.zeros((HID, dpad), _F32).at[:, :dout].set(w)
    bp = jnp.zeros((dpad,), _F32).at[:dout].set(b)
    return wp, bp


def kernel(graph_state, node_pos, ori_pos, rotate, enc, dec, pol):
    gs3 = graph_state.reshape(N, 64, 3)
    ox = gs3[:, :, 0]
    oy = gs3[:, :, 1]
    at = gs3[:, :, 2]
    rotf = rotate.reshape(N, 4)
    eps = jax.random.normal(jax.random.key(11), (N, 64), _F32)
    eps_s = jax.random.normal(jax.random.key(12), (N, 10, 2), _F32)
    noise = 0.1 * jax.random.normal(jax.random.key(13), (N, 2), _F32)

    nbr = _knn(node_pos)
    idx = nbr.T.reshape(-1)
    pos128 = jnp.zeros((N, HID), _F32).at[:, :2].set(node_pos)
    pg3 = _gather_rows(pos128, idx).reshape(K, N, HID)
    relx, rely = _edges(pg3, node_pos, rotf)

    h, y = _encin(graph_state, enc['W_in'], enc['b_in'],
                  enc['W_msg0'][:HID, :])
    _, y_enc = _graphnet_tail(h, y, relx, rely, idx, enc, 2,
                              enc['W_out'], enc['b_out'])

    wd = dec['W_in']
    wctx = wd[64:172].reshape(54, 2, HID)
    zpad = jnp.zeros((10, HID), _F32)
    wxd = jnp.concatenate([zpad, wctx[:, 0, :]], axis=0)
    wyd = jnp.concatenate([zpad, wctx[:, 1, :]], axis=0)
    h, y, klp = _decin(y_enc, eps, ox, oy, at, wd[:64], wxd, wyd,
                       wd[172:], dec['b_in'], dec['W_msg0'][:HID, :])
    wop, bop = _pad_out(dec['W_out'], dec['b_out'], 50)
    _, y_dec = _graphnet_tail(h, y, relx, rely, idx, dec, 2, wop, bop)
    kl = 0.5 * jnp.sum(klp[:, 0, 0]) / N

    rp = y_dec[:, :50].reshape(N, 10, 5)
    rrx, rry, ro, rrotf, rnp, lossp = _mid(rp, ox, oy, at, eps_s, noise,
                                           rotf, ori_pos, node_pos)
    rec_loss = jnp.sum(lossp[:, 0, 0]) / jnp.maximum(
        jnp.sum(lossp[:, 0, 1]), 1.0)

    nbr_p = _knn(rnp)
    idx_p = nbr_p.T.reshape(-1)
    rnp128 = jnp.zeros((N, HID), _F32).at[:, :2].set(rnp)
    pg3_p = _gather_rows(rnp128, idx_p).reshape(K, N, HID)
    relx_p, rely_p = _edges(pg3_p, rnp, rrotf)
    wp3 = pol['W_in'].reshape(64, 3, HID)
    h, y = _polin(rrx, rry, at, wp3[:, 0, :], wp3[:, 1, :], wp3[:, 2, :],
                  pol['b_in'], pol['W_msg0'][:HID, :])
    wopp, bopp = _pad_out(pol['W_out'], pol['b_out'], 250)
    _, y_pol = _graphnet_tail(h, y, relx_p, rely_p, idx_p, pol, 3,
                              wopp, bopp)
    action_preds = y_pol[:, :250]

    rec_rot = rrotf.reshape(N, 2, 2)
    return action_preds, rec_loss, kl, ro, rec_rot


# speedup vs baseline: 10.8253x; 1.0011x over previous
import functools

import jax
import jax.numpy as jnp
from jax import lax
from jax.experimental import pallas as pl
from jax.experimental.pallas import tpu as pltpu
from jax.experimental.pallas import tpu_sc as plsc

N = 2048
HID = 128
K = 16
BLK = 256
GRID = N // BLK
LOG2PI = 1.8378770664093453


def _row(d):
    return pl.BlockSpec((BLK, d), lambda i: (i, 0))


def _full(*shape):
    return pl.BlockSpec(shape, lambda i: tuple(0 for _ in shape))


_KMAJ = pl.BlockSpec((K, BLK, HID), lambda i: (0, i, 0))
_F32 = jnp.float32


def _bdot(a, b):
    return jnp.dot(a.astype(jnp.bfloat16), b.astype(jnp.bfloat16),
                   preferred_element_type=_F32)


def _bfr(x):
    return x.astype(jnp.bfloat16).astype(_F32)


def _gather_rows(table, idx):
    mesh = plsc.VectorSubcoreMesh(core_axis_name="c", subcore_axis_name="s")
    n_out = idx.shape[0]
    width = table.shape[1]
    per_w = n_out // 32
    n_chunk = per_w // 128

    @functools.partial(
        pl.kernel, mesh=mesh,
        out_type=jax.ShapeDtypeStruct((n_out, width), _F32),
        scratch_types=[
            pltpu.VMEM((128,), jnp.int32),
            pltpu.VMEM((128,), jnp.int32),
            pltpu.VMEM((128, width), _F32),
            pltpu.VMEM((128, width), _F32),
            pltpu.SemaphoreType.DMA,
            pltpu.SemaphoreType.DMA,
            pltpu.SemaphoreType.DMA,
            pltpu.SemaphoreType.DMA,
        ],
    )
    def k(table_hbm, idx_hbm, out_hbm, idx0, idx1, rows0, rows1,
          gs0, gs1, ws0, ws1):
        wid = lax.axis_index("s") * 2 + lax.axis_index("c")
        base = wid * per_w
        idxv = (idx0, idx1)
        rows = (rows0, rows1)
        gsem = (gs0, gs1)
        wsem = (ws0, ws1)
        g = [None, None]
        wb = [None, None]
        pltpu.sync_copy(idx_hbm.at[pl.ds(base, 128)], idxv[0])
        g[0] = pltpu.async_copy(table_hbm.at[idxv[0]], rows[0], gsem[0])
        for c in range(n_chunk):
            b = c % 2
            nb = (c + 1) % 2
            if c + 1 < n_chunk:
                if c + 1 >= 2:
                    wb[nb].wait()
                pltpu.sync_copy(
                    idx_hbm.at[pl.ds(base + (c + 1) * 128, 128)], idxv[nb])
                g[nb] = pltpu.async_copy(
                    table_hbm.at[idxv[nb]], rows[nb], gsem[nb])
            g[b].wait()
            wb[b] = pltpu.async_copy(
                rows[b], out_hbm.at[pl.ds(base + c * 128, 128)], wsem[b])
        wb[(n_chunk - 2) % 2].wait()
        wb[(n_chunk - 1) % 2].wait()

    return k(table, idx)


def _knn_body(pos_ref, posT_ref, nbr_ref):
    px = pos_ref[:, 0:1]
    py = pos_ref[:, 1:2]
    qx = posT_ref[0:1, :]
    qy = posT_ref[1:2, :]
    dx = qx - px
    dy = qy - py
    dist = jnp.sqrt(dx * dx + dy * dy + 1e-9)
    jj = lax.broadcasted_iota(jnp.int32, dist.shape, 1)
    idx_cols = []
    for k in range(K):
        m = jnp.min(dist, axis=1, keepdims=True)
        idx = jnp.min(jnp.where(dist == m, jj, N), axis=1, keepdims=True)
        idx_cols.append(idx)
        if k < K - 1:
            dist = jnp.where(jj == idx, jnp.inf, dist)
    nbr_ref[...] = jnp.concatenate(idx_cols, axis=1)


def _knn(pos):
    return pl.pallas_call(
        _knn_body, grid=(GRID,),
        in_specs=[_row(2), _full(2, N)],
        out_specs=pl.BlockSpec((BLK, K), lambda i: (i, 0)),
        out_shape=jax.ShapeDtypeStruct((N, K), jnp.int32),
    )(pos, pos.T)


def _edges_body(pg_ref, pos_ref, rot_ref, relx_ref, rely_ref):
    px = pos_ref[:, 0:1]
    py = pos_ref[:, 1:2]
    r00b = _bfr(rot_ref[:, 0:1])
    r01b = _bfr(rot_ref[:, 1:2])
    r10b = _bfr(rot_ref[:, 2:3])
    r11b = _bfr(rot_ref[:, 3:4])
    for k in range(K):
        dxb = _bfr(pg_ref[k, :, 0:1] - px)
        dyb = _bfr(pg_ref[k, :, 1:2] - py)
        relx_ref[:, k:k + 1] = _bfr(dxb * r00b + dyb * r10b)
        rely_ref[:, k:k + 1] = _bfr(dxb * r01b + dyb * r11b)


def _edges(pg3, pos, rotf):
    return pl.pallas_call(
        _edges_body, grid=(GRID,),
        in_specs=[pl.BlockSpec((K, BLK, 2), lambda i: (0, i, 0)),
                  _row(2), _row(4)],
        out_specs=[_row(K), _row(K)],
        out_shape=[jax.ShapeDtypeStruct((N, K), _F32),
                   jax.ShapeDtypeStruct((N, K), _F32)],
    )(pg3, pos, rotf)


def _encin_body(x_ref, wi_ref, bi_ref, wm_ref, h_ref, y_ref):
    h = jnp.maximum(
        _bdot(x_ref[...], wi_ref[...])
        + bi_ref[...], 0.0)
    h_ref[...] = h
    y_ref[...] = _bdot(h, wm_ref[...])


def _encin(x, wi, bi, wm):
    din = x.shape[1]
    return pl.pallas_call(
        _encin_body, grid=(GRID,),
        in_specs=[_row(din), _full(din, HID), _full(1, HID),
                  _full(HID, HID)],
        out_specs=[_row(HID), _row(HID)],
        out_shape=[jax.ShapeDtypeStruct((N, HID), _F32),
                   jax.ShapeDtypeStruct((N, HID), _F32)],
    )(x, wi, bi.reshape(1, HID), wm)


def _layer_body(h_ref, g3_ref, relx_ref, rely_ref, we_ref, be_ref,
                wb_ref, bm_ref, wu_ref, bu_ref,
                wn_ref, bn_ref, h_out_ref, y_ref):
    we0b = _bfr(we_ref[0:1, :])
    we1b = _bfr(we_ref[1:2, :])
    be = be_ref[...]
    e_ks = []
    for k in range(K):
        rx = relx_ref[:, k:k + 1]
        ry = rely_ref[:, k:k + 1]
        e_ks.append(jnp.maximum(rx * we0b + ry * we1b + be, 0.0))
    e = jnp.concatenate(e_ks, axis=0)
    g = g3_ref[...].reshape(K * BLK, HID)
    eb = _bdot(e, wb_ref[...])
    msg = jnp.maximum(g + eb + bm_ref[...], 0.0)
    m3 = msg.reshape(K, BLK, HID)
    acc = m3[0]
    for k in range(1, K):
        acc = acc + m3[k]
    agg = acc * (1.0 / K)
    h = h_ref[...]
    u = jnp.maximum(
        _bdot(h, wu_ref[:HID, :])
        + _bdot(agg, wu_ref[HID:, :])
        + bu_ref[...], 0.0)
    hn = h + u
    h_out_ref[...] = hn
    y_ref[...] = _bdot(hn, wn_ref[...]) \
        + bn_ref[...]


def _layer(h, g3, relx, rely, we, be, w_msg_bot, b_msg, w_upd, b_upd,
           w_next, b_next):
    dout = w_next.shape[1]
    return pl.pallas_call(
        _layer_body, grid=(GRID,),
        in_specs=[_row(HID), _KMAJ, _row(K), _row(K), _full(2, HID),
                  _full(1, HID), _full(HID, HID), _full(1, HID),
                  _full(2 * HID, HID), _full(1, HID), _full(HID, dout),
                  _full(1, dout)],
        out_specs=[_row(HID), _row(dout)],
        out_shape=[jax.ShapeDtypeStruct((N, HID), _F32),
                   jax.ShapeDtypeStruct((N, dout), _F32)],
    )(h, g3, relx, rely, we, be.reshape(1, HID), w_msg_bot,
      b_msg.reshape(1, HID), w_upd, b_upd.reshape(1, HID), w_next,
      b_next.reshape(1, dout))


def _decin_body(he_ref, eps_ref, ox_ref, oy_ref, at_ref, wz_ref, wx_ref,
                wy_ref, wa_ref, b_ref, wm_ref, h_ref, y_ref, klp_ref):
    he = he_ref[...]
    mu = he[:, :64]
    lv = he[:, 64:]
    elv = jnp.exp(lv)
    z = mu + eps_ref[...] * jnp.exp(lv * 0.5)
    klv = jnp.sum(elv + mu * mu - lv - 1.0)
    klp_ref[...] = jnp.broadcast_to(klv.reshape(1, 1, 1), (1, 1, HID))
    h0 = jnp.maximum(
        _bdot(z, wz_ref[...])
        + _bdot(ox_ref[...], wx_ref[...])
        + _bdot(oy_ref[...], wy_ref[...])
        + _bdot(at_ref[...], wa_ref[...])
        + b_ref[...], 0.0)
    h_ref[...] = h0
    y_ref[...] = _bdot(h0, wm_ref[...])


def _decin(he, eps, ox, oy, at, wz, wx, wy, wa, b, wm):
    return pl.pallas_call(
        _decin_body, grid=(GRID,),
        in_specs=[_row(HID), _row(64), _row(64), _row(64), _row(64),
                  _full(64, HID), _full(64, HID), _full(64, HID),
                  _full(64, HID), _full(1, HID), _full(HID, HID)],
        out_specs=[_row(HID), _row(HID),
                   pl.BlockSpec((1, 1, HID), lambda i: (i, 0, 0))],
        out_shape=[jax.ShapeDtypeStruct((N, HID), _F32),
                   jax.ShapeDtypeStruct((N, HID), _F32),
                   jax.ShapeDtypeStruct((GRID, 1, HID), _F32)],
    )(he, eps, ox, oy, at, wz, wx, wy, wa, b.reshape(1, HID), wm)


def _polin_body(rx_ref, ry_ref, at_ref, wx_ref, wy_ref, wa_ref, b_ref,
                wm_ref, h_ref, y_ref):
    h0 = jnp.maximum(
        _bdot(rx_ref[...], wx_ref[...])
        + _bdot(ry_ref[...], wy_ref[...])
        + _bdot(at_ref[...], wa_ref[...])
        + b_ref[...], 0.0)
    h_ref[...] = h0
    y_ref[...] = _bdot(h0, wm_ref[...])


def _polin(rx, ry, at, wx, wy, wa, b, wm):
    return pl.pallas_call(
        _polin_body, grid=(GRID,),
        in_specs=[_row(64), _row(64), _row(64), _full(64, HID),
                  _full(64, HID), _full(64, HID), _full(1, HID),
                  _full(HID, HID)],
        out_specs=[_row(HID), _row(HID)],
        out_shape=[jax.ShapeDtypeStruct((N, HID), _F32),
                   jax.ShapeDtypeStruct((N, HID), _F32)],
    )(rx, ry, at, wx, wy, wa, b.reshape(1, HID), wm)


def _mid_body(rp0_ref, rp1_ref, rp2_ref, rp3_ref, rp4_ref, ox_ref, oy_ref,
              at_ref, e0_ref, e1_ref, nz_ref, rot_ref, op_ref, np_ref,
              rrx_ref, rry_ref, ro_ref, rrot_ref, rnp_ref, lp_ref):
    ox = ox_ref[...]
    oy = oy_ref[...]
    at = at_ref[...]
    mx = rp0_ref[...]
    my = rp1_ref[...]
    off = rp2_ref[...]
    lp3 = rp3_ref[...]
    lp4 = rp4_ref[...]
    d0 = jnp.exp(lp3)
    d1 = jnp.exp(lp4)
    y0 = (ox[:, :10] - mx) / d0
    y1 = (oy[:, :10] - my - off * y0) / d1
    neg_logp = 0.5 * (y0 * y0 + y1 * y1) + lp3 + lp4 + LOG2PI
    mask10 = (at[:, :10] != 0.0).astype(_F32)
    nl_sum = jnp.sum(neg_logp * mask10).reshape(1, 1)
    m_sum = jnp.sum(mask10).reshape(1, 1)
    lp_ref[...] = jnp.concatenate(
        [nl_sum, m_sum, jnp.zeros((1, HID - 2), _F32)], axis=1
    ).reshape(1, 1, HID)
    e0 = e0_ref[...]
    e1 = e1_ref[...]
    s0 = (mx + d0 * e0) * mask10
    s1 = (my + off * e0 + d1 * e1) * mask10
    rsx = jnp.concatenate([s0, ox[:, 10:]], axis=1)
    rsy = jnp.concatenate([s1, oy[:, 10:]], axis=1)
    nx = nz_ref[:, 0:1]
    ny = nz_ref[:, 1:2]
    curx = rsx[:, 0:1] + nx
    cury = rsy[:, 0:1] + ny
    rpx = ox[:, 10:60]
    rpy = oy[:, 10:60]
    wdt = at[:, 10:60]
    ddx = rpx - curx
    ddy = rpy - cury
    avail = (wdt != 0.0).astype(_F32)
    dd = jnp.sqrt(ddx * ddx + ddy * ddy) - wdt - avail * 1000.0
    dda = dd.reshape(BLK, 50, 1)
    ddb = dd.reshape(BLK, 1, 50)
    iii = lax.broadcasted_iota(jnp.int32, (BLK, 50, 50), 1)
    jjj = lax.broadcasted_iota(jnp.int32, (BLK, 50, 50), 2)
    before = (ddb < dda) | ((ddb == dda) & (jjj < iii))
    rank = jnp.sum(before.astype(jnp.int32), axis=2)
    oh = (rank.reshape(BLK, 50, 1) == jjj).astype(_F32)
    sx = jnp.sum(oh * rpx.reshape(BLK, 50, 1), axis=1)
    sy = jnp.sum(oh * rpy.reshape(BLK, 50, 1), axis=1)
    sw = jnp.sum(oh * wdt.reshape(BLK, 50, 1), axis=1)
    rsx = jnp.concatenate([rsx[:, :10], sx, rsx[:, 60:]], axis=1)
    rsy = jnp.concatenate([rsy[:, :10], sy, rsy[:, 60:]], axis=1)
    c0 = rot_ref[:, 0:1]
    c1 = rot_ref[:, 1:2]
    c2 = rot_ref[:, 2:3]
    c3 = rot_ref[:, 3:4]
    opx = op_ref[:, 0:1]
    opy = op_ref[:, 1:2]
    rsxb = _bfr(rsx)
    rsyb = _bfr(rsy)
    c0b = _bfr(c0)
    c1b = _bfr(c1)
    c2b = _bfr(c2)
    c3b = _bfr(c3)
    absx = rsxb * c0b + rsyb * c1b + opx
    absy = rsxb * c2b + rsyb * c3b + opy
    nrx = _bfr(nx) * c0b + _bfr(ny) * c1b
    nry = _bfr(nx) * c2b + _bfr(ny) * c3b
    rox = absx[:, 0:1] + nrx
    roy = absy[:, 0:1] + nry
    gx = absx[:, 63:64] - rox
    gy = absy[:, 63:64] - roy
    r2 = gx * gx + gy * gy
    inv = lax.rsqrt(r2)
    cc = jnp.where(r2 > 0.0, gx * inv, 1.0)
    ss = jnp.where(r2 > 0.0, gy * inv, 0.0)
    ax = _bfr(absx - rox)
    ay = _bfr(absy - roy)
    ccb = _bfr(cc)
    ssb = _bfr(ss)
    rrx = ax * ccb + ay * ssb
    rry = ay * ccb - ax * ssb
    recattr = jnp.concatenate([at[:, :10], sw], axis=1)
    mask60 = (recattr != 0.0).astype(_F32)
    rrx = jnp.concatenate([rrx[:, :60] * mask60, rrx[:, 60:]], axis=1)
    rry = jnp.concatenate([rry[:, :60] * mask60, rry[:, 60:]], axis=1)
    rrx_ref[...] = rrx
    rry_ref[...] = rry
    ro_ref[...] = jnp.concatenate([rox, roy], axis=1)
    rrot_ref[...] = jnp.concatenate([cc, -ss, ss, cc], axis=1)
    npx = np_ref[:, 0:1]
    npy = np_ref[:, 1:2]
    rnp_ref[...] = jnp.concatenate(
        [npx + rox - opx, npy + roy - opy], axis=1)


def _mid(rp, ox, oy, at, eps_s, noise, rotf, ori_pos, node_pos):
    return pl.pallas_call(
        _mid_body, grid=(GRID,),
        in_specs=[_row(10)] * 5 + [_row(64)] * 3 + [_row(10)] * 2
        + [_row(2), _row(4), _row(2), _row(2)],
        out_specs=[_row(64), _row(64), _row(2), _row(4), _row(2),
                   pl.BlockSpec((1, 1, HID), lambda i: (i, 0, 0))],
        out_shape=[jax.ShapeDtypeStruct((N, 64), _F32),
                   jax.ShapeDtypeStruct((N, 64), _F32),
                   jax.ShapeDtypeStruct((N, 2), _F32),
                   jax.ShapeDtypeStruct((N, 4), _F32),
                   jax.ShapeDtypeStruct((N, 2), _F32),
                   jax.ShapeDtypeStruct((GRID, 1, HID), _F32)],
    )(rp[..., 0], rp[..., 1], rp[..., 2], rp[..., 3], rp[..., 4],
      ox, oy, at, eps_s[..., 0], eps_s[..., 1], noise, rotf, ori_pos,
      node_pos)


def _graphnet_tail(h, y, relx, rely, idx, p, n_layers, w_out_p, b_out_p):
    for l in range(n_layers):
        g3 = _gather_rows(y, idx).reshape(K, N, HID)
        if l == n_layers - 1:
            w_next, b_next = w_out_p, b_out_p
        else:
            w_next = p['W_msg%d' % (l + 1)][:HID, :]
            b_next = jnp.zeros((w_next.shape[1],), _F32)
        h, y = _layer(h, g3, relx, rely, p['W_e'], p['b_e'],
                      p['W_msg%d' % l][HID:, :], p['b_msg%d' % l],
                      p['W_upd%d' % l], p['b_upd%d' % l], w_next, b_next)
    return h, y


def _pad_out(w, b, dout):
    dpad = dout if dout % HID == 0 else (dout // HID + 1) * HID
    wp = jnp.zeros((HID, dpad), _F32).at[:, :dout].set(w)
    bp = jnp.zeros((dpad,), _F32).at[:dout].set(b)
    return wp, bp


def kernel(graph_state, node_pos, ori_pos, rotate, enc, dec, pol):
    gs3 = graph_state.reshape(N, 64, 3)
    ox = gs3[:, :, 0]
    oy = gs3[:, :, 1]
    at = gs3[:, :, 2]
    rotf = rotate.reshape(N, 4)
    eps = jax.random.normal(jax.random.key(11), (N, 64), _F32)
    eps_s = jax.random.normal(jax.random.key(12), (N, 10, 2), _F32)
    noise = 0.1 * jax.random.normal(jax.random.key(13), (N, 2), _F32)

    nbr = _knn(node_pos)
    idx = nbr.T.reshape(-1)
    pos128 = jnp.zeros((N, HID), _F32).at[:, :2].set(node_pos)
    pg3 = _gather_rows(pos128, idx)[:, :2].reshape(K, N, 2)
    relx, rely = _edges(pg3, node_pos, rotf)

    h, y = _encin(graph_state, enc['W_in'], enc['b_in'],
                  enc['W_msg0'][:HID, :])
    _, y_enc = _graphnet_tail(h, y, relx, rely, idx, enc, 2,
                              enc['W_out'], enc['b_out'])

    wd = dec['W_in']
    wctx = wd[64:172].reshape(54, 2, HID)
    zpad = jnp.zeros((10, HID), _F32)
    wxd = jnp.concatenate([zpad, wctx[:, 0, :]], axis=0)
    wyd = jnp.concatenate([zpad, wctx[:, 1, :]], axis=0)
    h, y, klp = _decin(y_enc, eps, ox, oy, at, wd[:64], wxd, wyd,
                       wd[172:], dec['b_in'], dec['W_msg0'][:HID, :])
    wop, bop = _pad_out(dec['W_out'], dec['b_out'], 50)
    _, y_dec = _graphnet_tail(h, y, relx, rely, idx, dec, 2, wop, bop)
    kl = 0.5 * jnp.sum(klp[:, 0, 0]) / N

    rp = y_dec[:, :50].reshape(N, 10, 5)
    rrx, rry, ro, rrotf, rnp, lossp = _mid(rp, ox, oy, at, eps_s, noise,
                                           rotf, ori_pos, node_pos)
    rec_loss = jnp.sum(lossp[:, 0, 0]) / jnp.maximum(
        jnp.sum(lossp[:, 0, 1]), 1.0)

    nbr_p = _knn(rnp)
    idx_p = nbr_p.T.reshape(-1)
    rnp128 = jnp.zeros((N, HID), _F32).at[:, :2].set(rnp)
    pg3_p = _gather_rows(rnp128, idx_p)[:, :2].reshape(K, N, 2)
    relx_p, rely_p = _edges(pg3_p, rnp, rrotf)
    wp3 = pol['W_in'].reshape(64, 3, HID)
    h, y = _polin(rrx, rry, at, wp3[:, 0, :], wp3[:, 1, :], wp3[:, 2, :],
                  pol['b_in'], pol['W_msg0'][:HID, :])
    wopp, bopp = _pad_out(pol['W_out'], pol['b_out'], 250)
    _, y_pol = _graphnet_tail(h, y, relx_p, rely_p, idx_p, pol, 3,
                              wopp, bopp)
    action_preds = y_pol[:, :250]

    rec_rot = rrotf.reshape(N, 2, 2)
    return action_preds, rec_loss, kl, ro, rec_rot
